# baseline (speedup 1.0000x reference)
import functools

import jax
import jax.numpy as jnp
from jax import lax
from jax.experimental import pallas as pl
from jax.experimental.pallas import tpu as pltpu
from jax.experimental.pallas import tpu_sc as plsc

N = 10000
NP = 10240
E = 160000
H = 256
NG = 256
CH = 128
EPT = 10112
NCHUNK = EPT // CH
EP = EPT * 16
RPT = NP // 16
RB = 512
GRID = NP // RB
F32 = jnp.float32


def _sc_mesh():
    return plsc.VectorSubcoreMesh(core_axis_name="c", subcore_axis_name="s",
                                  num_cores=2, num_subcores=16)


@functools.partial(
    pl.kernel,
    out_type=jax.ShapeDtypeStruct((NP, H), F32),
    mesh=_sc_mesh(),
    scratch_types=[
        pltpu.VMEM((64,), jnp.int32),
        pltpu.VMEM((64, H), F32),
        pltpu.SemaphoreType.DMA,
    ],
)
def _emb_gather(emb_hbm, wid_hbm, out_hbm, idx_v, rows_v, sem):
    wid = lax.axis_index("s") * 2 + lax.axis_index("c")
    base = wid * (NP // 32)

    def body(j, carry):
        b = base + j * 64
        pltpu.sync_copy(wid_hbm.at[pl.ds(b, 64)], idx_v)
        pltpu.async_copy(emb_hbm.at[idx_v], rows_v, sem).wait()
        pltpu.sync_copy(rows_v, out_hbm.at[pl.ds(b, 64)])
        return carry

    lax.fori_loop(0, (NP // 32) // 64, body, None)


def _make_edge_pass(nh):
    out_type = (
        [jax.ShapeDtypeStruct((NP, 128), F32) for _ in range(2 * nh)]
        + [jax.ShapeDtypeStruct((NP,), F32) for _ in range(nh)]
    )
    scratch = [
        pltpu.VMEM((4, CH), jnp.int32),
        pltpu.VMEM((2 * CH,), F32),
        pltpu.VMEM((2 * CH,), F32),
        pltpu.VMEM((2 * (CH + 16),), F32),
        pltpu.VMEM((2 * CH, 128), F32),
        pltpu.VMEM_SHARED((NP, 128), F32),
        pltpu.VMEM_SHARED((NP,), F32),
        pltpu.SemaphoreType.DMA((2,)),
        pltpu.SemaphoreType.DMA((2,)),
        pltpu.SemaphoreType.DMA((2,)),
    ]

    def body(*refs):
        sd, zrows0, zvec0 = refs[0:3]
        es = refs[3:3 + nh]
        ed = refs[3 + nh:3 + 2 * nh]
        zz = refs[3 + 2 * nh:3 + 4 * nh]
        o = 3 + 4 * nh
        numer = refs[o:o + 2 * nh]
        den = refs[o + 2 * nh:o + 3 * nh]
        (eidx, esv, edv, exv, zrows, acc, dacc,
         gsem, ssem, dsem) = refs[o + 3 * nh:]
        XB = CH + 16

        cid = lax.axis_index("c")
        tid = lax.axis_index("s")
        rbase = tid * RPT

        for h in range(nh):
            zref = [zz[2 * h], zz[2 * h + 1]]

            def issue(kk, b, h=h, zref=zref):
                ebase = tid * EPT + kk * CH
                pltpu.sync_copy(sd.at[:, pl.ds(ebase, CH)],
                                eidx.at[pl.ds(2 * b, 2)])
                pltpu.async_copy(es[h].at[eidx.at[2 * b]],
                                 esv.at[pl.ds(b * CH, CH)], gsem.at[b])
                pltpu.async_copy(ed[h].at[eidx.at[2 * b + 1]],
                                 edv.at[pl.ds(b * CH, CH)], gsem.at[b])

                @pl.when(cid == 0)
                def _():
                    pltpu.async_copy(zref[0].at[eidx.at[2 * b]],
                                     zrows.at[pl.ds(b * CH, CH)], gsem.at[b])

                @pl.when(cid == 1)
                def _():
                    pltpu.async_copy(zref[1].at[eidx.at[2 * b]],
                                     zrows.at[pl.ds(b * CH, CH)], gsem.at[b])

            def drain_gather(b, h=h, zref=zref):
                pltpu.make_async_copy(es[h].at[pl.ds(0, CH)],
                                      esv.at[pl.ds(b * CH, CH)],
                                      gsem.at[b]).wait()
                pltpu.make_async_copy(ed[h].at[pl.ds(0, CH)],
                                      edv.at[pl.ds(b * CH, CH)],
                                      gsem.at[b]).wait()
                pltpu.make_async_copy(zref[0].at[pl.ds(0, CH)],
                                      zrows.at[pl.ds(b * CH, CH)],
                                      gsem.at[b]).wait()

            def drain_scatter(b, h=h, zref=zref):
                pltpu.make_async_copy(zref[0].at[pl.ds(0, CH)],
                                      zrows.at[pl.ds(b * CH, CH)],
                                      ssem.at[b]).wait()

            def drain_den(b, h=h):
                pltpu.make_async_copy(es[h].at[pl.ds(0, CH)],
                                      exv.at[pl.ds(b * XB, CH)],
                                      dsem.at[b]).wait()

            pltpu.sync_copy(zrows0.at[pl.ds(rbase, RPT)], acc.at[pl.ds(rbase, RPT)])
            pltpu.sync_copy(zvec0.at[pl.ds(rbase, RPT)], dacc.at[pl.ds(rbase, RPT)])
            plsc.subcore_barrier()

            issue(0, 0)

            def chunk(kk, carry, h=h):
                b = lax.rem(kk, 2)
                nb = 1 - b

                @pl.when(kk + 1 < NCHUNK)
                def _():
                    @pl.when(kk >= 1)
                    def _():
                        drain_scatter(nb)

                        @pl.when(cid == 0)
                        def _():
                            drain_den(nb)

                    issue(kk + 1, nb)

                drain_gather(b)

                for j in range(CH // 16):
                    t = (esv[pl.ds(b * CH + 16 * j, 16)]
                         + edv[pl.ds(b * CH + 16 * j, 16)])
                    t = jnp.maximum(t, 0.01 * t)
                    exv[pl.ds(b * XB + 16 * j, 16)] = jnp.exp(t)

                @pl.when(cid == 0)
                def _():
                    pltpu.async_copy(exv.at[pl.ds(b * XB, CH)],
                                     dacc.at[eidx.at[2 * b + 1]], dsem.at[b],
                                     add=True)

                @plsc.parallel_loop(0, CH, unroll=8)
                def scale(e):
                    s = exv[pl.ds(b * XB + e, 16)][0]
                    r = b * CH + e
                    for j in range(8):
                        sl = pl.ds(16 * j, 16)
                        zrows[r, sl] = zrows[r, sl] * s
                pltpu.async_copy(zrows.at[pl.ds(b * CH, CH)],
                                 acc.at[eidx.at[2 * b + 1]], ssem.at[b],
                                 add=True)
                return carry

            lax.fori_loop(0, NCHUNK, chunk, None)
            drain_scatter(0)
            drain_scatter(1)

            @pl.when(cid == 0)
            def _():
                drain_den(0)
                drain_den(1)

            plsc.subcore_barrier()

            @pl.when(cid == 0)
            def _(h=h):
                pltpu.sync_copy(acc.at[pl.ds(rbase, RPT)],
                                numer[2 * h].at[pl.ds(rbase, RPT)])
                pltpu.sync_copy(dacc.at[pl.ds(rbase, RPT)],
                                den[h].at[pl.ds(rbase, RPT)])

            @pl.when(cid == 1)
            def _(h=h):
                pltpu.sync_copy(acc.at[pl.ds(rbase, RPT)],
                                numer[2 * h + 1].at[pl.ds(rbase, RPT)])

    return pl.kernel(body, out_type=out_type, mesh=_sc_mesh(),
                     scratch_types=scratch)


_edge_pass4 = _make_edge_pass(4)
_edge_pass1 = _make_edge_pass(1)


def _k2_project(h0p, wcat, aall, gidp):
    def kern(h0_ref, w_ref, a_ref, gid_ref, *outs):
        zs = outs[0:8]
        ev = outs[8:16]
        s0_ref, cnt_ref = outs[16], outs[17]
        i = pl.program_id(0)
        h0b = h0_ref[...]
        z = jnp.dot(h0b, w_ref[...], preferred_element_type=F32)
        eall = jnp.dot(z, a_ref[...], preferred_element_type=F32)
        for h in range(4):
            zs[2 * h][...] = z[:, 256 * h:256 * h + 128]
            zs[2 * h + 1][...] = z[:, 256 * h + 128:256 * (h + 1)]
            ev[h][...] = eall[:, 2 * h]
            ev[4 + h][...] = eall[:, 2 * h + 1]
        gi = lax.broadcasted_iota(jnp.int32, (NG, RB), 0)
        gf = (gi == gid_ref[...][None, :]).astype(F32)
        s0c = jnp.dot(gf, h0b, preferred_element_type=F32)
        cntc = jnp.sum(gf, axis=1)

        @pl.when(i == 0)
        def _():
            s0_ref[...] = jnp.zeros_like(s0_ref)
            cnt_ref[...] = jnp.zeros_like(cnt_ref)

        s0_ref[...] += s0c
        cnt_ref[...] += cntc

    zspec = pl.BlockSpec((RB, 128), lambda i: (i, 0))
    vspec = pl.BlockSpec((RB,), lambda i: (i,))
    return pl.pallas_call(
        kern,
        grid=(GRID,),
        in_specs=[
            pl.BlockSpec((RB, H), lambda i: (i, 0)),
            pl.BlockSpec((H, 4 * H), lambda i: (0, 0)),
            pl.BlockSpec((4 * H, 8), lambda i: (0, 0)),
            pl.BlockSpec((RB,), lambda i: (i,)),
        ],
        out_specs=[zspec] * 8 + [vspec] * 8 + [
            pl.BlockSpec((NG, H), lambda i: (0, 0)),
            pl.BlockSpec((NG,), lambda i: (0,)),
        ],
        out_shape=[jax.ShapeDtypeStruct((NP, 128), F32)] * 8
        + [jax.ShapeDtypeStruct((NP,), F32)] * 8
        + [jax.ShapeDtypeStruct((NG, H), F32),
           jax.ShapeDtypeStruct((NG,), F32)],
    )(h0p, wcat, aall, gidp)


def _stats(nums, dens):
    nh = len(dens)

    def kern(*refs):
        nrefs = refs[0:2 * nh]
        drefs = refs[2 * nh:3 * nh]
        sum_ref, ssq_ref = refs[3 * nh], refs[3 * nh + 1]
        i = pl.program_id(0)
        ridx = i * RB + lax.broadcasted_iota(jnp.int32, (RB, 128), 0)
        valid = (ridx < N).astype(F32)
        srows, qrows = [], []
        for h in range(nh):
            d = jnp.maximum(drefs[h][...], 1e-16)
            for half in range(2):
                x = jnp.maximum(nrefs[2 * h + half][...] / d[:, None], 0.0)
                x = x * valid
                srows.append(jnp.sum(x, axis=0))
                qrows.append(jnp.sum(x * x, axis=0))
        scur = jnp.stack(srows, axis=0)
        qcur = jnp.stack(qrows, axis=0)

        @pl.when(i == 0)
        def _():
            sum_ref[...] = jnp.zeros_like(sum_ref)
            ssq_ref[...] = jnp.zeros_like(ssq_ref)

        sum_ref[...] += scur
        ssq_ref[...] += qcur

    nspec = pl.BlockSpec((RB, 128), lambda i: (i, 0))
    dspec = pl.BlockSpec((RB,), lambda i: (i,))
    sspec = pl.BlockSpec((2 * nh, 128), lambda i: (0, 0))
    return pl.pallas_call(
        kern,
        grid=(GRID,),
        in_specs=[nspec] * (2 * nh) + [dspec] * nh,
        out_specs=[sspec, sspec],
        out_shape=[jax.ShapeDtypeStruct((2 * nh, 128), F32)] * 2,
    )(*nums, *dens)


def _norm(nums, dens, ssum, ssq, gmat, bmat):
    nh = len(dens)
    pieces = []
    for h in range(nh):
        d = jnp.maximum(dens[h], 1e-16)
        for half in range(2):
            x = jnp.maximum(nums[2 * h + half] / d[:, None], 0.0)
            mu = ssum[2 * h + half, :] * (1.0 / N)
            var = ssq[2 * h + half, :] * (1.0 / N) - mu * mu
            xn = (x - mu[None, :]) * lax.rsqrt(var + 1e-5)[None, :]
            pieces.append(xn * gmat[2 * h + half, :][None, :]
                          + bmat[2 * h + half, :][None, :])
    return jnp.concatenate(pieces, axis=1)


def _k5_mid(nums, dens, ssum, ssq, wo, ao, gmat, bmat):
    def kern(*refs):
        nrefs = refs[0:8]
        drefs = refs[8:12]
        sum_ref, ssq_ref, wo_ref, ao_ref, g_ref, b_ref = refs[12:18]
        zol, zoh, eso, edo = refs[18:22]
        hcat = _norm([r[...] for r in nrefs], [r[...] for r in drefs],
                     sum_ref[...], ssq_ref[...], g_ref[...], b_ref[...])
        zo = jnp.dot(hcat, wo_ref[...], preferred_element_type=F32)
        eo = jnp.dot(zo, ao_ref[...], preferred_element_type=F32)
        zol[...] = zo[:, :128]
        zoh[...] = zo[:, 128:]
        eso[...] = eo[:, 0]
        edo[...] = eo[:, 1]

    nspec = pl.BlockSpec((RB, 128), lambda i: (i, 0))
    dspec = pl.BlockSpec((RB,), lambda i: (i,))
    return pl.pallas_call(
        kern,
        grid=(GRID,),
        in_specs=[nspec] * 8 + [dspec] * 4 + [
            pl.BlockSpec((8, 128), lambda i: (0, 0)),
            pl.BlockSpec((8, 128), lambda i: (0, 0)),
            pl.BlockSpec((4 * H, H), lambda i: (0, 0)),
            pl.BlockSpec((H, 2), lambda i: (0, 0)),
            pl.BlockSpec((8, 128), lambda i: (0, 0)),
            pl.BlockSpec((8, 128), lambda i: (0, 0)),
        ],
        out_specs=[nspec, nspec, dspec, dspec],
        out_shape=[jax.ShapeDtypeStruct((NP, 128), F32)] * 2
        + [jax.ShapeDtypeStruct((NP,), F32)] * 2,
    )(*nums, *dens, ssum, ssq, wo, ao, gmat, bmat)


def _k7_readout(numol, numoh, deno, ssum, ssq, gmat, bmat, gidp):
    def kern(nl, nh_, dr, sum_ref, ssq_ref, g_ref, b_ref, gid_ref, s1_ref):
        i = pl.program_id(0)
        hfin = _norm([nl[...], nh_[...]], [dr[...]],
                     sum_ref[...], ssq_ref[...], g_ref[...], b_ref[...])
        gi = lax.broadcasted_iota(jnp.int32, (NG, RB), 0)
        gf = (gi == gid_ref[...][None, :]).astype(F32)
        s1c = jnp.dot(gf, hfin, preferred_element_type=F32)

        @pl.when(i == 0)
        def _():
            s1_ref[...] = jnp.zeros_like(s1_ref)

        s1_ref[...] += s1c

    return pl.pallas_call(
        kern,
        grid=(GRID,),
        in_specs=[
            pl.BlockSpec((RB, 128), lambda i: (i, 0)),
            pl.BlockSpec((RB, 128), lambda i: (i, 0)),
            pl.BlockSpec((RB,), lambda i: (i,)),
            pl.BlockSpec((2, 128), lambda i: (0, 0)),
            pl.BlockSpec((2, 128), lambda i: (0, 0)),
            pl.BlockSpec((2, 128), lambda i: (0, 0)),
            pl.BlockSpec((2, 128), lambda i: (0, 0)),
            pl.BlockSpec((RB,), lambda i: (i,)),
        ],
        out_specs=pl.BlockSpec((NG, H), lambda i: (0, 0)),
        out_shape=jax.ShapeDtypeStruct((NG, H), F32),
    )(numol, numoh, deno, ssum, ssq, gmat, bmat, gidp)


def _k8_lstm(s0, s1, cnt, p):
    def kern(s0_ref, s1_ref, cnt_ref, wih0, whh0, bih0, bhh0,
             wih1, whh1, bih1, bhh1, out_ref):
        c = jnp.maximum(cnt_ref[...], 1.0)
        x0 = s0_ref[...] / c[:, None]
        x1 = s1_ref[...] / c[:, None]
        inp = [x0, x1]
        wih = [wih0, wih1]
        whh = [whh0, whh1]
        bih = [bih0, bih1]
        bhh = [bhh0, bhh1]
        dn = (((1,), (1,)), ((), ()))
        hfs = []
        for l in range(2):
            h = jnp.zeros((NG, H), F32)
            cc = jnp.zeros((NG, H), F32)
            outs = []
            for t in range(2):
                gates = (lax.dot_general(inp[t], wih[l][...], dn,
                                         preferred_element_type=F32)
                         + bih[l][...][None, :]
                         + lax.dot_general(h, whh[l][...], dn,
                                           preferred_element_type=F32)
                         + bhh[l][...][None, :])
                i_ = jax.nn.sigmoid(gates[:, 0:H])
                f_ = jax.nn.sigmoid(gates[:, H:2 * H])
                g_ = jnp.tanh(gates[:, 2 * H:3 * H])
                o_ = jax.nn.sigmoid(gates[:, 3 * H:4 * H])
                cc = f_ * cc + i_ * g_
                h = o_ * jnp.tanh(cc)
                outs.append(h)
            inp = outs
            hfs.append(h)
        out_ref[...] = hfs[0] + hfs[1]

    return pl.pallas_call(
        kern,
        out_shape=jax.ShapeDtypeStruct((NG, H), F32),
    )(s0, s1, cnt, p['Wih0'], p['Whh0'], p['bih0'], p['bhh0'],
      p['Wih1'], p['Whh1'], p['bih1'], p['bhh1'])


def kernel(params, wid, edge_index, graph_ids):
    p = params
    i32 = jnp.int32
    src = edge_index[0].astype(i32)
    dst = edge_index[1].astype(i32)

    widp = jnp.concatenate([wid.astype(i32), jnp.zeros((NP - N,), i32)])
    gidp = jnp.concatenate([graph_ids.astype(i32),
                            jnp.full((NP - N,), NG + 40, i32)])
    pidx = (N + (jnp.arange(EP - E, dtype=i32) % (NP - N))).astype(i32)
    sd = jnp.stack([jnp.concatenate([src, pidx]),
                    jnp.concatenate([dst, pidx])])
    zrows0 = jnp.zeros((NP, 128), F32)
    zvec0 = jnp.zeros((NP,), F32)

    h0p = _emb_gather(p['emb'], widp)

    wcat = jnp.concatenate([p['W%d' % i] for i in range(4)], axis=1)
    aall = jnp.zeros((4 * H, 8), F32)
    for h in range(4):
        aall = aall.at[256 * h:256 * (h + 1), 2 * h].set(p['a%d' % h][:H])
        aall = aall.at[256 * h:256 * (h + 1), 2 * h + 1].set(p['a%d' % h][H:])

    k2 = _k2_project(h0p, wcat, aall, gidp)
    zs = k2[0:8]
    evs = k2[8:16]
    s0, cnt = k2[16], k2[17]

    ep4 = _edge_pass4(sd, zrows0, zvec0,
                      evs[0], evs[1], evs[2], evs[3],
                      evs[4], evs[5], evs[6], evs[7],
                      zs[0], zs[1], zs[2], zs[3],
                      zs[4], zs[5], zs[6], zs[7])
    nums = ep4[0:8]
    dens = ep4[8:12]

    ssum, ssq = _stats(nums, dens)

    gmat = jnp.concatenate([p['g%d' % h].reshape(2, 128) for h in range(4)])
    bmat = jnp.concatenate([p['b%d' % h].reshape(2, 128) for h in range(4)])
    ao = jnp.stack([p['ao'][:H], p['ao'][H:]], axis=1)

    zol, zoh, eso, edo = _k5_mid(nums, dens, ssum, ssq, p['Wo'], ao,
                                 gmat, bmat)

    ep1 = _edge_pass1(sd, zrows0, zvec0, eso, edo, zol, zoh)
    numol, numoh, deno = ep1

    ssum2, ssq2 = _stats([numol, numoh], [deno])
    s1 = _k7_readout(numol, numoh, deno, ssum2, ssq2,
                     p['go'].reshape(2, 128), p['bo'].reshape(2, 128), gidp)

    return _k8_lstm(s0, s1, cnt, p)


# speedup vs baseline: 1.0032x; 1.0032x over previous
import functools

import jax
import jax.numpy as jnp
from jax import lax
from jax.experimental import pallas as pl
from jax.experimental.pallas import tpu as pltpu
from jax.experimental.pallas import tpu_sc as plsc

N = 10000
NP = 10240
E = 160000
H = 256
NG = 256
CH = 128
EPT = 10112
NCHUNK = EPT // CH
EP = EPT * 16
RPT = NP // 16
RB = 512
GRID = NP // RB
F32 = jnp.float32


def _sc_mesh():
    return plsc.VectorSubcoreMesh(core_axis_name="c", subcore_axis_name="s",
                                  num_cores=2, num_subcores=16)


@functools.partial(
    pl.kernel,
    out_type=jax.ShapeDtypeStruct((NP, H), F32),
    mesh=_sc_mesh(),
    scratch_types=[
        pltpu.VMEM((64,), jnp.int32),
        pltpu.VMEM((64, H), F32),
        pltpu.SemaphoreType.DMA,
    ],
)
def _emb_gather(emb_hbm, wid_hbm, out_hbm, idx_v, rows_v, sem):
    wid = lax.axis_index("s") * 2 + lax.axis_index("c")
    base = wid * (NP // 32)

    def body(j, carry):
        b = base + j * 64
        pltpu.sync_copy(wid_hbm.at[pl.ds(b, 64)], idx_v)
        pltpu.async_copy(emb_hbm.at[idx_v], rows_v, sem).wait()
        pltpu.sync_copy(rows_v, out_hbm.at[pl.ds(b, 64)])
        return carry

    lax.fori_loop(0, (NP // 32) // 64, body, None)


def _make_edge_pass(nh):
    out_type = (
        [jax.ShapeDtypeStruct((NP, 128), F32) for _ in range(2 * nh)]
        + [jax.ShapeDtypeStruct((NP,), F32) for _ in range(nh)]
    )
    scratch = [
        pltpu.VMEM((4, CH), jnp.int32),
        pltpu.VMEM((2 * CH,), F32),
        pltpu.VMEM((2 * CH,), F32),
        pltpu.VMEM((2 * (CH + 16),), F32),
        pltpu.VMEM((2 * CH, 128), F32),
        pltpu.VMEM_SHARED((NP, 128), F32),
        pltpu.VMEM_SHARED((NP,), F32),
        pltpu.SemaphoreType.DMA((2,)),
        pltpu.SemaphoreType.DMA((2,)),
        pltpu.SemaphoreType.DMA((2,)),
    ]

    def body(*refs):
        sd, zrows0, zvec0 = refs[0:3]
        es = refs[3:3 + nh]
        ed = refs[3 + nh:3 + 2 * nh]
        zz = refs[3 + 2 * nh:3 + 4 * nh]
        o = 3 + 4 * nh
        numer = refs[o:o + 2 * nh]
        den = refs[o + 2 * nh:o + 3 * nh]
        (eidx, esv, edv, exv, zrows, acc, dacc,
         gsem, ssem, dsem) = refs[o + 3 * nh:]
        XB = CH + 16

        cid = lax.axis_index("c")
        tid = lax.axis_index("s")
        rbase = tid * RPT

        for h in range(nh):
            zref = [zz[2 * h], zz[2 * h + 1]]
            dcore = h % 2

            def issue(kk, b, h=h, zref=zref):
                ebase = tid * EPT + kk * CH
                pltpu.sync_copy(sd.at[:, pl.ds(ebase, CH)],
                                eidx.at[pl.ds(2 * b, 2)])
                pltpu.async_copy(es[h].at[eidx.at[2 * b]],
                                 esv.at[pl.ds(b * CH, CH)], gsem.at[b])
                pltpu.async_copy(ed[h].at[eidx.at[2 * b + 1]],
                                 edv.at[pl.ds(b * CH, CH)], gsem.at[b])

                @pl.when(cid == 0)
                def _():
                    pltpu.async_copy(zref[0].at[eidx.at[2 * b]],
                                     zrows.at[pl.ds(b * CH, CH)], gsem.at[b])

                @pl.when(cid == 1)
                def _():
                    pltpu.async_copy(zref[1].at[eidx.at[2 * b]],
                                     zrows.at[pl.ds(b * CH, CH)], gsem.at[b])

            def drain_gather(b, h=h, zref=zref):
                pltpu.make_async_copy(es[h].at[pl.ds(0, CH)],
                                      esv.at[pl.ds(b * CH, CH)],
                                      gsem.at[b]).wait()
                pltpu.make_async_copy(ed[h].at[pl.ds(0, CH)],
                                      edv.at[pl.ds(b * CH, CH)],
                                      gsem.at[b]).wait()
                pltpu.make_async_copy(zref[0].at[pl.ds(0, CH)],
                                      zrows.at[pl.ds(b * CH, CH)],
                                      gsem.at[b]).wait()

            def drain_scatter(b, h=h, zref=zref):
                pltpu.make_async_copy(zref[0].at[pl.ds(0, CH)],
                                      zrows.at[pl.ds(b * CH, CH)],
                                      ssem.at[b]).wait()

            def drain_den(b, h=h):
                pltpu.make_async_copy(es[h].at[pl.ds(0, CH)],
                                      exv.at[pl.ds(b * XB, CH)],
                                      dsem.at[b]).wait()

            pltpu.sync_copy(zrows0.at[pl.ds(rbase, RPT)], acc.at[pl.ds(rbase, RPT)])
            pltpu.sync_copy(zvec0.at[pl.ds(rbase, RPT)], dacc.at[pl.ds(rbase, RPT)])
            plsc.subcore_barrier()

            issue(0, 0)

            def chunk(kk, carry, h=h):
                b = lax.rem(kk, 2)
                nb = 1 - b

                @pl.when(kk + 1 < NCHUNK)
                def _():
                    @pl.when(kk >= 1)
                    def _():
                        drain_scatter(nb)

                        @pl.when(cid == dcore)
                        def _():
                            drain_den(nb)

                    issue(kk + 1, nb)

                drain_gather(b)

                for j in range(CH // 16):
                    t = (esv[pl.ds(b * CH + 16 * j, 16)]
                         + edv[pl.ds(b * CH + 16 * j, 16)])
                    t = jnp.maximum(t, 0.01 * t)
                    exv[pl.ds(b * XB + 16 * j, 16)] = jnp.exp(t)

                @pl.when(cid == dcore)
                def _():
                    pltpu.async_copy(exv.at[pl.ds(b * XB, CH)],
                                     dacc.at[eidx.at[2 * b + 1]], dsem.at[b],
                                     add=True)

                @plsc.parallel_loop(0, CH, unroll=8)
                def scale(e):
                    s = exv[pl.ds(b * XB + e, 16)][0]
                    r = b * CH + e
                    for j in range(8):
                        sl = pl.ds(16 * j, 16)
                        zrows[r, sl] = zrows[r, sl] * s
                pltpu.async_copy(zrows.at[pl.ds(b * CH, CH)],
                                 acc.at[eidx.at[2 * b + 1]], ssem.at[b],
                                 add=True)
                return carry

            lax.fori_loop(0, NCHUNK, chunk, None)
            drain_scatter(0)
            drain_scatter(1)

            @pl.when(cid == dcore)
            def _():
                drain_den(0)
                drain_den(1)

            plsc.subcore_barrier()

            @pl.when(cid == 0)
            def _(h=h):
                pltpu.sync_copy(acc.at[pl.ds(rbase, RPT)],
                                numer[2 * h].at[pl.ds(rbase, RPT)])

            @pl.when(cid == 1)
            def _(h=h):
                pltpu.sync_copy(acc.at[pl.ds(rbase, RPT)],
                                numer[2 * h + 1].at[pl.ds(rbase, RPT)])

            @pl.when(cid == dcore)
            def _(h=h):
                pltpu.sync_copy(dacc.at[pl.ds(rbase, RPT)],
                                den[h].at[pl.ds(rbase, RPT)])

    return pl.kernel(body, out_type=out_type, mesh=_sc_mesh(),
                     scratch_types=scratch)


_edge_pass4 = _make_edge_pass(4)
_edge_pass1 = _make_edge_pass(1)


def _k2_project(h0p, wcat, aall, gidp):
    def kern(h0_ref, w_ref, a_ref, gid_ref, *outs):
        zs = outs[0:8]
        ev = outs[8:16]
        s0_ref, cnt_ref = outs[16], outs[17]
        i = pl.program_id(0)
        h0b = h0_ref[...]
        z = jnp.dot(h0b, w_ref[...], preferred_element_type=F32)
        eall = jnp.dot(z, a_ref[...], preferred_element_type=F32)
        for h in range(4):
            zs[2 * h][...] = z[:, 256 * h:256 * h + 128]
            zs[2 * h + 1][...] = z[:, 256 * h + 128:256 * (h + 1)]
            ev[h][...] = eall[:, 2 * h]
            ev[4 + h][...] = eall[:, 2 * h + 1]
        gi = lax.broadcasted_iota(jnp.int32, (NG, RB), 0)
        gf = (gi == gid_ref[...][None, :]).astype(F32)
        s0c = jnp.dot(gf, h0b, preferred_element_type=F32)
        cntc = jnp.sum(gf, axis=1)

        @pl.when(i == 0)
        def _():
            s0_ref[...] = jnp.zeros_like(s0_ref)
            cnt_ref[...] = jnp.zeros_like(cnt_ref)

        s0_ref[...] += s0c
        cnt_ref[...] += cntc

    zspec = pl.BlockSpec((RB, 128), lambda i: (i, 0))
    vspec = pl.BlockSpec((RB,), lambda i: (i,))
    return pl.pallas_call(
        kern,
        grid=(GRID,),
        in_specs=[
            pl.BlockSpec((RB, H), lambda i: (i, 0)),
            pl.BlockSpec((H, 4 * H), lambda i: (0, 0)),
            pl.BlockSpec((4 * H, 8), lambda i: (0, 0)),
            pl.BlockSpec((RB,), lambda i: (i,)),
        ],
        out_specs=[zspec] * 8 + [vspec] * 8 + [
            pl.BlockSpec((NG, H), lambda i: (0, 0)),
            pl.BlockSpec((NG,), lambda i: (0,)),
        ],
        out_shape=[jax.ShapeDtypeStruct((NP, 128), F32)] * 8
        + [jax.ShapeDtypeStruct((NP,), F32)] * 8
        + [jax.ShapeDtypeStruct((NG, H), F32),
           jax.ShapeDtypeStruct((NG,), F32)],
    )(h0p, wcat, aall, gidp)


def _stats(nums, dens):
    nh = len(dens)

    def kern(*refs):
        nrefs = refs[0:2 * nh]
        drefs = refs[2 * nh:3 * nh]
        sum_ref, ssq_ref = refs[3 * nh], refs[3 * nh + 1]
        i = pl.program_id(0)
        ridx = i * RB + lax.broadcasted_iota(jnp.int32, (RB, 128), 0)
        valid = (ridx < N).astype(F32)
        srows, qrows = [], []
        for h in range(nh):
            d = jnp.maximum(drefs[h][...], 1e-16)
            for half in range(2):
                x = jnp.maximum(nrefs[2 * h + half][...] / d[:, None], 0.0)
                x = x * valid
                srows.append(jnp.sum(x, axis=0))
                qrows.append(jnp.sum(x * x, axis=0))
        scur = jnp.stack(srows, axis=0)
        qcur = jnp.stack(qrows, axis=0)

        @pl.when(i == 0)
        def _():
            sum_ref[...] = jnp.zeros_like(sum_ref)
            ssq_ref[...] = jnp.zeros_like(ssq_ref)

        sum_ref[...] += scur
        ssq_ref[...] += qcur

    nspec = pl.BlockSpec((RB, 128), lambda i: (i, 0))
    dspec = pl.BlockSpec((RB,), lambda i: (i,))
    sspec = pl.BlockSpec((2 * nh, 128), lambda i: (0, 0))
    return pl.pallas_call(
        kern,
        grid=(GRID,),
        in_specs=[nspec] * (2 * nh) + [dspec] * nh,
        out_specs=[sspec, sspec],
        out_shape=[jax.ShapeDtypeStruct((2 * nh, 128), F32)] * 2,
    )(*nums, *dens)


def _norm(nums, dens, ssum, ssq, gmat, bmat):
    nh = len(dens)
    pieces = []
    for h in range(nh):
        d = jnp.maximum(dens[h], 1e-16)
        for half in range(2):
            x = jnp.maximum(nums[2 * h + half] / d[:, None], 0.0)
            mu = ssum[2 * h + half, :] * (1.0 / N)
            var = ssq[2 * h + half, :] * (1.0 / N) - mu * mu
            xn = (x - mu[None, :]) * lax.rsqrt(var + 1e-5)[None, :]
            pieces.append(xn * gmat[2 * h + half, :][None, :]
                          + bmat[2 * h + half, :][None, :])
    return jnp.concatenate(pieces, axis=1)


def _k5_mid(nums, dens, wo, ao, gmat, bmat):
    def kern(*refs):
        nrefs = refs[0:8]
        drefs = refs[8:12]
        wo_ref, ao_ref, g_ref, b_ref = refs[12:16]
        zol, zoh, eso, edo = refs[16:20]
        sum_ref, ssq_ref = refs[20:22]
        i = pl.program_id(0)
        ii = lax.rem(i, GRID)

        @pl.when(i < GRID)
        def _():
            ridx = ii * RB + lax.broadcasted_iota(jnp.int32, (RB, 128), 0)
            valid = (ridx < N).astype(F32)
            srows, qrows = [], []
            for h in range(4):
                d = jnp.maximum(drefs[h][...], 1e-16)
                for half in range(2):
                    x = jnp.maximum(nrefs[2 * h + half][...] / d[:, None], 0.0)
                    x = x * valid
                    srows.append(jnp.sum(x, axis=0))
                    qrows.append(jnp.sum(x * x, axis=0))
            scur = jnp.stack(srows, axis=0)
            qcur = jnp.stack(qrows, axis=0)

            @pl.when(i == 0)
            def _():
                sum_ref[...] = jnp.zeros_like(sum_ref)
                ssq_ref[...] = jnp.zeros_like(ssq_ref)

            sum_ref[...] += scur
            ssq_ref[...] += qcur

        @pl.when(i >= GRID)
        def _():
            hcat = _norm([r[...] for r in nrefs], [r[...] for r in drefs],
                         sum_ref[...], ssq_ref[...], g_ref[...], b_ref[...])
            zo = jnp.dot(hcat, wo_ref[...], preferred_element_type=F32)
            eo = jnp.dot(zo, ao_ref[...], preferred_element_type=F32)
            zol[...] = zo[:, :128]
            zoh[...] = zo[:, 128:]
            eso[...] = eo[:, 0]
            edo[...] = eo[:, 1]

    nspec = pl.BlockSpec((RB, 128), lambda i: (lax.rem(i, GRID), 0))
    dspec = pl.BlockSpec((RB,), lambda i: (lax.rem(i, GRID),))
    return pl.pallas_call(
        kern,
        grid=(2 * GRID,),
        in_specs=[nspec] * 8 + [dspec] * 4 + [
            pl.BlockSpec((4 * H, H), lambda i: (0, 0)),
            pl.BlockSpec((H, 2), lambda i: (0, 0)),
            pl.BlockSpec((8, 128), lambda i: (0, 0)),
            pl.BlockSpec((8, 128), lambda i: (0, 0)),
        ],
        out_specs=[nspec, nspec, dspec, dspec],
        out_shape=[jax.ShapeDtypeStruct((NP, 128), F32)] * 2
        + [jax.ShapeDtypeStruct((NP,), F32)] * 2,
        scratch_shapes=[pltpu.VMEM((8, 128), F32), pltpu.VMEM((8, 128), F32)],
    )(*nums, *dens, wo, ao, gmat, bmat)


def _k7_readout(numol, numoh, deno, gmat, bmat, gidp):
    def kern(nl, nh_, dr, g_ref, b_ref, gid_ref, s1_ref, sum_ref, ssq_ref):
        i = pl.program_id(0)
        ii = lax.rem(i, GRID)

        @pl.when(i < GRID)
        def _():
            ridx = ii * RB + lax.broadcasted_iota(jnp.int32, (RB, 128), 0)
            valid = (ridx < N).astype(F32)
            d = jnp.maximum(dr[...], 1e-16)
            srows, qrows = [], []
            for half in range(2):
                x = jnp.maximum([nl, nh_][half][...] / d[:, None], 0.0)
                x = x * valid
                srows.append(jnp.sum(x, axis=0))
                qrows.append(jnp.sum(x * x, axis=0))

            @pl.when(i == 0)
            def _():
                sum_ref[...] = jnp.zeros_like(sum_ref)
                ssq_ref[...] = jnp.zeros_like(ssq_ref)

            sum_ref[...] += jnp.stack(srows, axis=0)
            ssq_ref[...] += jnp.stack(qrows, axis=0)

        @pl.when(i >= GRID)
        def _():
            hfin = _norm([nl[...], nh_[...]], [dr[...]],
                         sum_ref[...], ssq_ref[...], g_ref[...], b_ref[...])
            gi = lax.broadcasted_iota(jnp.int32, (NG, RB), 0)
            gf = (gi == gid_ref[...][None, :]).astype(F32)
            s1c = jnp.dot(gf, hfin, preferred_element_type=F32)

            @pl.when(i == GRID)
            def _():
                s1_ref[...] = jnp.zeros_like(s1_ref)

            s1_ref[...] += s1c

    return pl.pallas_call(
        kern,
        grid=(2 * GRID,),
        in_specs=[
            pl.BlockSpec((RB, 128), lambda i: (lax.rem(i, GRID), 0)),
            pl.BlockSpec((RB, 128), lambda i: (lax.rem(i, GRID), 0)),
            pl.BlockSpec((RB,), lambda i: (lax.rem(i, GRID),)),
            pl.BlockSpec((2, 128), lambda i: (0, 0)),
            pl.BlockSpec((2, 128), lambda i: (0, 0)),
            pl.BlockSpec((RB,), lambda i: (lax.rem(i, GRID),)),
        ],
        out_specs=pl.BlockSpec((NG, H), lambda i: (0, 0)),
        out_shape=jax.ShapeDtypeStruct((NG, H), F32),
        scratch_shapes=[pltpu.VMEM((2, 128), F32), pltpu.VMEM((2, 128), F32)],
    )(numol, numoh, deno, gmat, bmat, gidp)


def _k8_lstm(s0, s1, cnt, p):
    def kern(s0_ref, s1_ref, cnt_ref, wih0, whh0, bih0, bhh0,
             wih1, whh1, bih1, bhh1, out_ref):
        c = jnp.maximum(cnt_ref[...], 1.0)
        x0 = s0_ref[...] / c[:, None]
        x1 = s1_ref[...] / c[:, None]
        inp = [x0, x1]
        wih = [wih0, wih1]
        whh = [whh0, whh1]
        bih = [bih0, bih1]
        bhh = [bhh0, bhh1]
        dn = (((1,), (1,)), ((), ()))
        hfs = []
        for l in range(2):
            h = jnp.zeros((NG, H), F32)
            cc = jnp.zeros((NG, H), F32)
            outs = []
            for t in range(2):
                gates = (lax.dot_general(inp[t], wih[l][...], dn,
                                         preferred_element_type=F32)
                         + bih[l][...][None, :]
                         + lax.dot_general(h, whh[l][...], dn,
                                           preferred_element_type=F32)
                         + bhh[l][...][None, :])
                i_ = jax.nn.sigmoid(gates[:, 0:H])
                f_ = jax.nn.sigmoid(gates[:, H:2 * H])
                g_ = jnp.tanh(gates[:, 2 * H:3 * H])
                o_ = jax.nn.sigmoid(gates[:, 3 * H:4 * H])
                cc = f_ * cc + i_ * g_
                h = o_ * jnp.tanh(cc)
                outs.append(h)
            inp = outs
            hfs.append(h)
        out_ref[...] = hfs[0] + hfs[1]

    return pl.pallas_call(
        kern,
        out_shape=jax.ShapeDtypeStruct((NG, H), F32),
    )(s0, s1, cnt, p['Wih0'], p['Whh0'], p['bih0'], p['bhh0'],
      p['Wih1'], p['Whh1'], p['bih1'], p['bhh1'])


def kernel(params, wid, edge_index, graph_ids):
    p = params
    i32 = jnp.int32
    src = edge_index[0].astype(i32)
    dst = edge_index[1].astype(i32)

    widp = jnp.concatenate([wid.astype(i32), jnp.zeros((NP - N,), i32)])
    gidp = jnp.concatenate([graph_ids.astype(i32),
                            jnp.full((NP - N,), NG + 40, i32)])
    pidx = (N + (jnp.arange(EP - E, dtype=i32) % (NP - N))).astype(i32)
    sd = jnp.stack([jnp.concatenate([src, pidx]),
                    jnp.concatenate([dst, pidx])])
    zrows0 = jnp.zeros((NP, 128), F32)
    zvec0 = jnp.zeros((NP,), F32)

    h0p = _emb_gather(p['emb'], widp)

    wcat = jnp.concatenate([p['W%d' % i] for i in range(4)], axis=1)
    aall = jnp.zeros((4 * H, 8), F32)
    for h in range(4):
        aall = aall.at[256 * h:256 * (h + 1), 2 * h].set(p['a%d' % h][:H])
        aall = aall.at[256 * h:256 * (h + 1), 2 * h + 1].set(p['a%d' % h][H:])

    k2 = _k2_project(h0p, wcat, aall, gidp)
    zs = k2[0:8]
    evs = k2[8:16]
    s0, cnt = k2[16], k2[17]

    ep4 = _edge_pass4(sd, zrows0, zvec0,
                      evs[0], evs[1], evs[2], evs[3],
                      evs[4], evs[5], evs[6], evs[7],
                      zs[0], zs[1], zs[2], zs[3],
                      zs[4], zs[5], zs[6], zs[7])
    nums = ep4[0:8]
    dens = ep4[8:12]

    gmat = jnp.concatenate([p['g%d' % h].reshape(2, 128) for h in range(4)])
    bmat = jnp.concatenate([p['b%d' % h].reshape(2, 128) for h in range(4)])
    ao = jnp.stack([p['ao'][:H], p['ao'][H:]], axis=1)

    zol, zoh, eso, edo = _k5_mid(nums, dens, p['Wo'], ao, gmat, bmat)

    ep1 = _edge_pass1(sd, zrows0, zvec0, eso, edo, zol, zoh)
    numol, numoh, deno = ep1

    s1 = _k7_readout(numol, numoh, deno,
                     p['go'].reshape(2, 128), p['bo'].reshape(2, 128), gidp)

    return _k8_lstm(s0, s1, cnt, p)


# speedup vs baseline: 1.1472x; 1.1435x over previous
import functools

import jax
import jax.numpy as jnp
from jax import lax
from jax.experimental import pallas as pl
from jax.experimental.pallas import tpu as pltpu
from jax.experimental.pallas import tpu_sc as plsc

N = 10000
NP = 10240
E = 160000
H = 256
NG = 256
CH = 128
EPT = 10112
NCHUNK = EPT // CH
EP = EPT * 16
RPT = NP // 16
RB = 512
GRID = NP // RB
F32 = jnp.float32


def _sc_mesh():
    return plsc.VectorSubcoreMesh(core_axis_name="c", subcore_axis_name="s",
                                  num_cores=2, num_subcores=16)


@functools.partial(
    pl.kernel,
    out_type=jax.ShapeDtypeStruct((NP, H), F32),
    mesh=_sc_mesh(),
    scratch_types=[
        pltpu.VMEM((64,), jnp.int32),
        pltpu.VMEM((64, H), F32),
        pltpu.SemaphoreType.DMA,
    ],
)
def _emb_gather(emb_hbm, wid_hbm, out_hbm, idx_v, rows_v, sem):
    wid = lax.axis_index("s") * 2 + lax.axis_index("c")
    base = wid * (NP // 32)

    def body(j, carry):
        b = base + j * 64
        pltpu.sync_copy(wid_hbm.at[pl.ds(b, 64)], idx_v)
        pltpu.async_copy(emb_hbm.at[idx_v], rows_v, sem).wait()
        pltpu.sync_copy(rows_v, out_hbm.at[pl.ds(b, 64)])
        return carry

    lax.fori_loop(0, (NP // 32) // 64, body, None)


def _make_edge_pass(nh):
    out_type = (
        [jax.ShapeDtypeStruct((NP, 128), F32) for _ in range(2 * nh)]
        + [jax.ShapeDtypeStruct((NP,), F32) for _ in range(nh)]
    )
    scratch = [
        pltpu.VMEM((6, CH), jnp.int32),
        pltpu.VMEM((2 * CH,), F32),
        pltpu.VMEM((2 * CH,), F32),
        pltpu.VMEM((2 * (CH + 16),), F32),
        pltpu.VMEM((2 * CH, 128), F32),
        pltpu.VMEM_SHARED((NP, 128), F32),
        pltpu.VMEM_SHARED((NP,), F32),
        pltpu.SemaphoreType.DMA((2,)),
        pltpu.SemaphoreType.DMA((2,)),
        pltpu.SemaphoreType.DMA((2,)),
        pltpu.SemaphoreType.DMA((3,)),
    ]

    def body(*refs):
        sd, zrows0, zvec0 = refs[0:3]
        es = refs[3:3 + nh]
        ed = refs[3 + nh:3 + 2 * nh]
        zz = refs[3 + 2 * nh:3 + 4 * nh]
        o = 3 + 4 * nh
        numer = refs[o:o + 2 * nh]
        den = refs[o + 2 * nh:o + 3 * nh]
        (eidx, esv, edv, exv, zrows, acc, dacc,
         gsem, ssem, dsem, esem) = refs[o + 3 * nh:]
        XB = CH + 16

        cid = lax.axis_index("c")
        tid = lax.axis_index("s")
        rbase = tid * RPT

        def issue_eidx(kk, g):
            pltpu.async_copy(sd.at[:, pl.ds(tid * EPT + kk * CH, CH)],
                             eidx.at[pl.ds(2 * g, 2)], esem.at[g])

        def wait_eidx(g):
            pltpu.make_async_copy(sd.at[:, pl.ds(0, CH)],
                                  eidx.at[pl.ds(2 * g, 2)], esem.at[g]).wait()

        for h in range(nh):
            zref = [zz[2 * h], zz[2 * h + 1]]
            dcore = h % 2

            def issue(b, g, h=h, zref=zref):
                pltpu.async_copy(es[h].at[eidx.at[2 * g]],
                                 esv.at[pl.ds(b * CH, CH)], gsem.at[b])
                pltpu.async_copy(ed[h].at[eidx.at[2 * g + 1]],
                                 edv.at[pl.ds(b * CH, CH)], gsem.at[b])

                @pl.when(cid == 0)
                def _():
                    pltpu.async_copy(zref[0].at[eidx.at[2 * g]],
                                     zrows.at[pl.ds(b * CH, CH)], gsem.at[b])

                @pl.when(cid == 1)
                def _():
                    pltpu.async_copy(zref[1].at[eidx.at[2 * g]],
                                     zrows.at[pl.ds(b * CH, CH)], gsem.at[b])

            def drain_gather(b, h=h, zref=zref):
                pltpu.make_async_copy(es[h].at[pl.ds(0, CH)],
                                      esv.at[pl.ds(b * CH, CH)],
                                      gsem.at[b]).wait()
                pltpu.make_async_copy(ed[h].at[pl.ds(0, CH)],
                                      edv.at[pl.ds(b * CH, CH)],
                                      gsem.at[b]).wait()
                pltpu.make_async_copy(zref[0].at[pl.ds(0, CH)],
                                      zrows.at[pl.ds(b * CH, CH)],
                                      gsem.at[b]).wait()

            def drain_scatter(b, h=h, zref=zref):
                pltpu.make_async_copy(zref[0].at[pl.ds(0, CH)],
                                      zrows.at[pl.ds(b * CH, CH)],
                                      ssem.at[b]).wait()

            def drain_den(b, h=h):
                pltpu.make_async_copy(es[h].at[pl.ds(0, CH)],
                                      exv.at[pl.ds(b * XB, CH)],
                                      dsem.at[b]).wait()

            pltpu.sync_copy(zrows0.at[pl.ds(rbase, RPT)], acc.at[pl.ds(rbase, RPT)])
            pltpu.sync_copy(zvec0.at[pl.ds(rbase, RPT)], dacc.at[pl.ds(rbase, RPT)])
            plsc.subcore_barrier()

            issue_eidx(0, 0)
            issue_eidx(1, 1)
            wait_eidx(0)
            issue(0, 0)

            def chunk(kk, carry, h=h):
                b = lax.rem(kk, 2)
                nb = 1 - b
                g = lax.rem(kk, 3)
                gn = lax.rem(kk + 1, 3)
                g2 = lax.rem(kk + 2, 3)

                @pl.when(kk + 1 < NCHUNK)
                def _():
                    @pl.when(kk >= 1)
                    def _():
                        drain_scatter(nb)

                        @pl.when(cid == dcore)
                        def _():
                            drain_den(nb)

                    @pl.when(kk + 2 < NCHUNK)
                    def _():
                        issue_eidx(kk + 2, g2)

                    wait_eidx(gn)
                    issue(nb, gn)

                drain_gather(b)

                for j in range(CH // 16):
                    t = (esv[pl.ds(b * CH + 16 * j, 16)]
                         + edv[pl.ds(b * CH + 16 * j, 16)])
                    t = jnp.maximum(t, 0.01 * t)
                    exv[pl.ds(b * XB + 16 * j, 16)] = jnp.exp(t)

                @pl.when(cid == dcore)
                def _():
                    pltpu.async_copy(exv.at[pl.ds(b * XB, CH)],
                                     dacc.at[eidx.at[2 * g + 1]], dsem.at[b],
                                     add=True)

                @plsc.parallel_loop(0, CH, unroll=8)
                def scale(e):
                    s = exv[pl.ds(b * XB + e, 16)][0]
                    r = b * CH + e
                    for j in range(8):
                        sl = pl.ds(16 * j, 16)
                        zrows[r, sl] = zrows[r, sl] * s
                pltpu.async_copy(zrows.at[pl.ds(b * CH, CH)],
                                 acc.at[eidx.at[2 * g + 1]], ssem.at[b],
                                 add=True)
                return carry

            lax.fori_loop(0, NCHUNK, chunk, None)
            drain_scatter(0)
            drain_scatter(1)

            @pl.when(cid == dcore)
            def _():
                drain_den(0)
                drain_den(1)

            plsc.subcore_barrier()

            @pl.when(cid == 0)
            def _(h=h):
                pltpu.sync_copy(acc.at[pl.ds(rbase, RPT)],
                                numer[2 * h].at[pl.ds(rbase, RPT)])

            @pl.when(cid == 1)
            def _(h=h):
                pltpu.sync_copy(acc.at[pl.ds(rbase, RPT)],
                                numer[2 * h + 1].at[pl.ds(rbase, RPT)])

            @pl.when(cid == dcore)
            def _(h=h):
                pltpu.sync_copy(dacc.at[pl.ds(rbase, RPT)],
                                den[h].at[pl.ds(rbase, RPT)])

    return pl.kernel(body, out_type=out_type, mesh=_sc_mesh(),
                     scratch_types=scratch)


_edge_pass4 = _make_edge_pass(4)
_edge_pass1 = _make_edge_pass(1)


def _k2_project(h0p, wcat, aall, gidp):
    def kern(h0_ref, w_ref, a_ref, gid_ref, *outs):
        zs = outs[0:8]
        ev = outs[8:16]
        s0_ref, cnt_ref = outs[16], outs[17]
        i = pl.program_id(0)
        h0b = h0_ref[...]
        z = jnp.dot(h0b, w_ref[...], preferred_element_type=F32)
        eall = jnp.dot(z, a_ref[...], preferred_element_type=F32)
        for h in range(4):
            zs[2 * h][...] = z[:, 256 * h:256 * h + 128]
            zs[2 * h + 1][...] = z[:, 256 * h + 128:256 * (h + 1)]
            ev[h][...] = eall[:, 2 * h]
            ev[4 + h][...] = eall[:, 2 * h + 1]
        gi = lax.broadcasted_iota(jnp.int32, (NG, RB), 0)
        gf = (gi == gid_ref[...][None, :]).astype(F32)
        s0c = jnp.dot(gf, h0b, preferred_element_type=F32)
        cntc = jnp.sum(gf, axis=1)

        @pl.when(i == 0)
        def _():
            s0_ref[...] = jnp.zeros_like(s0_ref)
            cnt_ref[...] = jnp.zeros_like(cnt_ref)

        s0_ref[...] += s0c
        cnt_ref[...] += cntc

    zspec = pl.BlockSpec((RB, 128), lambda i: (i, 0))
    vspec = pl.BlockSpec((RB,), lambda i: (i,))
    return pl.pallas_call(
        kern,
        grid=(GRID,),
        in_specs=[
            pl.BlockSpec((RB, H), lambda i: (i, 0)),
            pl.BlockSpec((H, 4 * H), lambda i: (0, 0)),
            pl.BlockSpec((4 * H, 8), lambda i: (0, 0)),
            pl.BlockSpec((RB,), lambda i: (i,)),
        ],
        out_specs=[zspec] * 8 + [vspec] * 8 + [
            pl.BlockSpec((NG, H), lambda i: (0, 0)),
            pl.BlockSpec((NG,), lambda i: (0,)),
        ],
        out_shape=[jax.ShapeDtypeStruct((NP, 128), F32)] * 8
        + [jax.ShapeDtypeStruct((NP,), F32)] * 8
        + [jax.ShapeDtypeStruct((NG, H), F32),
           jax.ShapeDtypeStruct((NG,), F32)],
    )(h0p, wcat, aall, gidp)


def _stats(nums, dens):
    nh = len(dens)

    def kern(*refs):
        nrefs = refs[0:2 * nh]
        drefs = refs[2 * nh:3 * nh]
        sum_ref, ssq_ref = refs[3 * nh], refs[3 * nh + 1]
        i = pl.program_id(0)
        ridx = i * RB + lax.broadcasted_iota(jnp.int32, (RB, 128), 0)
        valid = (ridx < N).astype(F32)
        srows, qrows = [], []
        for h in range(nh):
            d = jnp.maximum(drefs[h][...], 1e-16)
            for half in range(2):
                x = jnp.maximum(nrefs[2 * h + half][...] / d[:, None], 0.0)
                x = x * valid
                srows.append(jnp.sum(x, axis=0))
                qrows.append(jnp.sum(x * x, axis=0))
        scur = jnp.stack(srows, axis=0)
        qcur = jnp.stack(qrows, axis=0)

        @pl.when(i == 0)
        def _():
            sum_ref[...] = jnp.zeros_like(sum_ref)
            ssq_ref[...] = jnp.zeros_like(ssq_ref)

        sum_ref[...] += scur
        ssq_ref[...] += qcur

    nspec = pl.BlockSpec((RB, 128), lambda i: (i, 0))
    dspec = pl.BlockSpec((RB,), lambda i: (i,))
    sspec = pl.BlockSpec((2 * nh, 128), lambda i: (0, 0))
    return pl.pallas_call(
        kern,
        grid=(GRID,),
        in_specs=[nspec] * (2 * nh) + [dspec] * nh,
        out_specs=[sspec, sspec],
        out_shape=[jax.ShapeDtypeStruct((2 * nh, 128), F32)] * 2,
    )(*nums, *dens)


def _norm(nums, dens, ssum, ssq, gmat, bmat):
    nh = len(dens)
    pieces = []
    for h in range(nh):
        d = jnp.maximum(dens[h], 1e-16)
        for half in range(2):
            x = jnp.maximum(nums[2 * h + half] / d[:, None], 0.0)
            mu = ssum[2 * h + half, :] * (1.0 / N)
            var = ssq[2 * h + half, :] * (1.0 / N) - mu * mu
            xn = (x - mu[None, :]) * lax.rsqrt(var + 1e-5)[None, :]
            pieces.append(xn * gmat[2 * h + half, :][None, :]
                          + bmat[2 * h + half, :][None, :])
    return jnp.concatenate(pieces, axis=1)


def _k5_mid(nums, dens, wo, ao, gmat, bmat):
    def kern(*refs):
        nrefs = refs[0:8]
        drefs = refs[8:12]
        wo_ref, ao_ref, g_ref, b_ref = refs[12:16]
        zol, zoh, eso, edo = refs[16:20]
        sum_ref, ssq_ref = refs[20:22]
        i = pl.program_id(0)
        ii = lax.rem(i, GRID)

        @pl.when(i < GRID)
        def _():
            ridx = ii * RB + lax.broadcasted_iota(jnp.int32, (RB, 128), 0)
            valid = (ridx < N).astype(F32)
            srows, qrows = [], []
            for h in range(4):
                d = jnp.maximum(drefs[h][...], 1e-16)
                for half in range(2):
                    x = jnp.maximum(nrefs[2 * h + half][...] / d[:, None], 0.0)
                    x = x * valid
                    srows.append(jnp.sum(x, axis=0))
                    qrows.append(jnp.sum(x * x, axis=0))
            scur = jnp.stack(srows, axis=0)
            qcur = jnp.stack(qrows, axis=0)

            @pl.when(i == 0)
            def _():
                sum_ref[...] = jnp.zeros_like(sum_ref)
                ssq_ref[...] = jnp.zeros_like(ssq_ref)

            sum_ref[...] += scur
            ssq_ref[...] += qcur

        @pl.when(i >= GRID)
        def _():
            hcat = _norm([r[...] for r in nrefs], [r[...] for r in drefs],
                         sum_ref[...], ssq_ref[...], g_ref[...], b_ref[...])
            zo = jnp.dot(hcat, wo_ref[...], preferred_element_type=F32)
            eo = jnp.dot(zo, ao_ref[...], preferred_element_type=F32)
            zol[...] = zo[:, :128]
            zoh[...] = zo[:, 128:]
            eso[...] = eo[:, 0]
            edo[...] = eo[:, 1]

    nspec = pl.BlockSpec((RB, 128), lambda i: (lax.rem(i, GRID), 0))
    dspec = pl.BlockSpec((RB,), lambda i: (lax.rem(i, GRID),))
    return pl.pallas_call(
        kern,
        grid=(2 * GRID,),
        in_specs=[nspec] * 8 + [dspec] * 4 + [
            pl.BlockSpec((4 * H, H), lambda i: (0, 0)),
            pl.BlockSpec((H, 2), lambda i: (0, 0)),
            pl.BlockSpec((8, 128), lambda i: (0, 0)),
            pl.BlockSpec((8, 128), lambda i: (0, 0)),
        ],
        out_specs=[nspec, nspec, dspec, dspec],
        out_shape=[jax.ShapeDtypeStruct((NP, 128), F32)] * 2
        + [jax.ShapeDtypeStruct((NP,), F32)] * 2,
        scratch_shapes=[pltpu.VMEM((8, 128), F32), pltpu.VMEM((8, 128), F32)],
    )(*nums, *dens, wo, ao, gmat, bmat)


def _k7_readout(numol, numoh, deno, gmat, bmat, gidp):
    def kern(nl, nh_, dr, g_ref, b_ref, gid_ref, s1_ref, sum_ref, ssq_ref):
        i = pl.program_id(0)
        ii = lax.rem(i, GRID)

        @pl.when(i < GRID)
        def _():
            ridx = ii * RB + lax.broadcasted_iota(jnp.int32, (RB, 128), 0)
            valid = (ridx < N).astype(F32)
            d = jnp.maximum(dr[...], 1e-16)
            srows, qrows = [], []
            for half in range(2):
                x = jnp.maximum([nl, nh_][half][...] / d[:, None], 0.0)
                x = x * valid
                srows.append(jnp.sum(x, axis=0))
                qrows.append(jnp.sum(x * x, axis=0))

            @pl.when(i == 0)
            def _():
                sum_ref[...] = jnp.zeros_like(sum_ref)
                ssq_ref[...] = jnp.zeros_like(ssq_ref)

            sum_ref[...] += jnp.stack(srows, axis=0)
            ssq_ref[...] += jnp.stack(qrows, axis=0)

        @pl.when(i >= GRID)
        def _():
            hfin = _norm([nl[...], nh_[...]], [dr[...]],
                         sum_ref[...], ssq_ref[...], g_ref[...], b_ref[...])
            gi = lax.broadcasted_iota(jnp.int32, (NG, RB), 0)
            gf = (gi == gid_ref[...][None, :]).astype(F32)
            s1c = jnp.dot(gf, hfin, preferred_element_type=F32)

            @pl.when(i == GRID)
            def _():
                s1_ref[...] = jnp.zeros_like(s1_ref)

            s1_ref[...] += s1c

    return pl.pallas_call(
        kern,
        grid=(2 * GRID,),
        in_specs=[
            pl.BlockSpec((RB, 128), lambda i: (lax.rem(i, GRID), 0)),
            pl.BlockSpec((RB, 128), lambda i: (lax.rem(i, GRID), 0)),
            pl.BlockSpec((RB,), lambda i: (lax.rem(i, GRID),)),
            pl.BlockSpec((2, 128), lambda i: (0, 0)),
            pl.BlockSpec((2, 128), lambda i: (0, 0)),
            pl.BlockSpec((RB,), lambda i: (lax.rem(i, GRID),)),
        ],
        out_specs=pl.BlockSpec((NG, H), lambda i: (0, 0)),
        out_shape=jax.ShapeDtypeStruct((NG, H), F32),
        scratch_shapes=[pltpu.VMEM((2, 128), F32), pltpu.VMEM((2, 128), F32)],
    )(numol, numoh, deno, gmat, bmat, gidp)


def _k8_lstm(s0, s1, cnt, p):
    def kern(s0_ref, s1_ref, cnt_ref, wih0, whh0, bih0, bhh0,
             wih1, whh1, bih1, bhh1, out_ref):
        c = jnp.maximum(cnt_ref[...], 1.0)
        x0 = s0_ref[...] / c[:, None]
        x1 = s1_ref[...] / c[:, None]
        inp = [x0, x1]
        wih = [wih0, wih1]
        whh = [whh0, whh1]
        bih = [bih0, bih1]
        bhh = [bhh0, bhh1]
        dn = (((1,), (1,)), ((), ()))
        hfs = []
        for l in range(2):
            h = jnp.zeros((NG, H), F32)
            cc = jnp.zeros((NG, H), F32)
            outs = []
            for t in range(2):
                gates = (lax.dot_general(inp[t], wih[l][...], dn,
                                         preferred_element_type=F32)
                         + bih[l][...][None, :]
                         + lax.dot_general(h, whh[l][...], dn,
                                           preferred_element_type=F32)
                         + bhh[l][...][None, :])
                i_ = jax.nn.sigmoid(gates[:, 0:H])
                f_ = jax.nn.sigmoid(gates[:, H:2 * H])
                g_ = jnp.tanh(gates[:, 2 * H:3 * H])
                o_ = jax.nn.sigmoid(gates[:, 3 * H:4 * H])
                cc = f_ * cc + i_ * g_
                h = o_ * jnp.tanh(cc)
                outs.append(h)
            inp = outs
            hfs.append(h)
        out_ref[...] = hfs[0] + hfs[1]

    return pl.pallas_call(
        kern,
        out_shape=jax.ShapeDtypeStruct((NG, H), F32),
    )(s0, s1, cnt, p['Wih0'], p['Whh0'], p['bih0'], p['bhh0'],
      p['Wih1'], p['Whh1'], p['bih1'], p['bhh1'])


def kernel(params, wid, edge_index, graph_ids):
    p = params
    i32 = jnp.int32
    src = edge_index[0].astype(i32)
    dst = edge_index[1].astype(i32)

    widp = jnp.concatenate([wid.astype(i32), jnp.zeros((NP - N,), i32)])
    gidp = jnp.concatenate([graph_ids.astype(i32),
                            jnp.full((NP - N,), NG + 40, i32)])
    pidx = (N + (jnp.arange(EP - E, dtype=i32) % (NP - N))).astype(i32)
    sd = jnp.stack([jnp.concatenate([src, pidx]),
                    jnp.concatenate([dst, pidx])])
    zrows0 = jnp.zeros((NP, 128), F32)
    zvec0 = jnp.zeros((NP,), F32)

    h0p = _emb_gather(p['emb'], widp)

    wcat = jnp.concatenate([p['W%d' % i] for i in range(4)], axis=1)
    aall = jnp.zeros((4 * H, 8), F32)
    for h in range(4):
        aall = aall.at[256 * h:256 * (h + 1), 2 * h].set(p['a%d' % h][:H])
        aall = aall.at[256 * h:256 * (h + 1), 2 * h + 1].set(p['a%d' % h][H:])

    k2 = _k2_project(h0p, wcat, aall, gidp)
    zs = k2[0:8]
    evs = k2[8:16]
    s0, cnt = k2[16], k2[17]

    ep4 = _edge_pass4(sd, zrows0, zvec0,
                      evs[0], evs[1], evs[2], evs[3],
                      evs[4], evs[5], evs[6], evs[7],
                      zs[0], zs[1], zs[2], zs[3],
                      zs[4], zs[5], zs[6], zs[7])
    nums = ep4[0:8]
    dens = ep4[8:12]

    gmat = jnp.concatenate([p['g%d' % h].reshape(2, 128) for h in range(4)])
    bmat = jnp.concatenate([p['b%d' % h].reshape(2, 128) for h in range(4)])
    ao = jnp.stack([p['ao'][:H], p['ao'][H:]], axis=1)

    zol, zoh, eso, edo = _k5_mid(nums, dens, p['Wo'], ao, gmat, bmat)

    ep1 = _edge_pass1(sd, zrows0, zvec0, eso, edo, zol, zoh)
    numol, numoh, deno = ep1

    s1 = _k7_readout(numol, numoh, deno,
                     p['go'].reshape(2, 128), p['bo'].reshape(2, 128), gidp)

    return _k8_lstm(s0, s1, cnt, p)


# speedup vs baseline: 1.1481x; 1.0008x over previous
import functools

import jax
import jax.numpy as jnp
from jax import lax
from jax.experimental import pallas as pl
from jax.experimental.pallas import tpu as pltpu
from jax.experimental.pallas import tpu_sc as plsc

N = 10000
NP = 10240
E = 160000
H = 256
NG = 256
CH = 128
EPT = 10112
NCHUNK = EPT // CH
EP = EPT * 16
RPT = NP // 16
RB = 512
GRID = NP // RB
F32 = jnp.float32


def _sc_mesh():
    return plsc.VectorSubcoreMesh(core_axis_name="c", subcore_axis_name="s",
                                  num_cores=2, num_subcores=16)


@functools.partial(
    pl.kernel,
    out_type=jax.ShapeDtypeStruct((NP, H), F32),
    mesh=_sc_mesh(),
    scratch_types=[
        pltpu.VMEM((64,), jnp.int32),
        pltpu.VMEM((64, H), F32),
        pltpu.SemaphoreType.DMA,
    ],
)
def _emb_gather(emb_hbm, wid_hbm, out_hbm, idx_v, rows_v, sem):
    wid = lax.axis_index("s") * 2 + lax.axis_index("c")
    base = wid * (NP // 32)

    def body(j, carry):
        b = base + j * 64
        pltpu.sync_copy(wid_hbm.at[pl.ds(b, 64)], idx_v)
        pltpu.async_copy(emb_hbm.at[idx_v], rows_v, sem).wait()
        pltpu.sync_copy(rows_v, out_hbm.at[pl.ds(b, 64)])
        return carry

    lax.fori_loop(0, (NP // 32) // 64, body, None)


def _make_edge_pass(nh):
    out_type = (
        [jax.ShapeDtypeStruct((NP, 128), F32) for _ in range(2 * nh)]
        + [jax.ShapeDtypeStruct((NP,), F32) for _ in range(nh)]
    )
    scratch = [
        pltpu.VMEM((6, CH), jnp.int32),
        pltpu.VMEM((2 * CH,), F32),
        pltpu.VMEM((2 * CH,), F32),
        pltpu.VMEM((2 * (CH + 16),), F32),
        pltpu.VMEM((2 * CH, 128), F32),
        pltpu.VMEM_SHARED((NP, 128), F32),
        pltpu.VMEM_SHARED((NP,), F32),
        pltpu.SemaphoreType.DMA((2,)),
        pltpu.SemaphoreType.DMA((2,)),
        pltpu.SemaphoreType.DMA((2,)),
        pltpu.SemaphoreType.DMA((3,)),
    ]

    def body(*refs):
        sd, zrows0, zvec0 = refs[0:3]
        es = refs[3:3 + nh]
        ed = refs[3 + nh:3 + 2 * nh]
        zz = refs[3 + 2 * nh:3 + 4 * nh]
        o = 3 + 4 * nh
        numer = refs[o:o + 2 * nh]
        den = refs[o + 2 * nh:o + 3 * nh]
        (eidx, esv, edv, exv, zrows, acc, dacc,
         gsem, ssem, dsem, esem) = refs[o + 3 * nh:]
        XB = CH + 16

        cid = lax.axis_index("c")
        tid = lax.axis_index("s")
        rbase = tid * RPT

        def issue_eidx(kk, g):
            pltpu.async_copy(sd.at[:, pl.ds(tid * EPT + kk * CH, CH)],
                             eidx.at[pl.ds(2 * g, 2)], esem.at[g])

        def wait_eidx(g):
            pltpu.make_async_copy(sd.at[:, pl.ds(0, CH)],
                                  eidx.at[pl.ds(2 * g, 2)], esem.at[g]).wait()

        for h in range(nh):
            zref = [zz[2 * h], zz[2 * h + 1]]
            dcore = h % 2

            def issue(b, g, h=h, zref=zref):
                pltpu.async_copy(es[h].at[eidx.at[2 * g]],
                                 esv.at[pl.ds(b * CH, CH)], gsem.at[b])
                pltpu.async_copy(ed[h].at[eidx.at[2 * g + 1]],
                                 edv.at[pl.ds(b * CH, CH)], gsem.at[b])

                @pl.when(cid == 0)
                def _():
                    pltpu.async_copy(zref[0].at[eidx.at[2 * g]],
                                     zrows.at[pl.ds(b * CH, CH)], gsem.at[b])

                @pl.when(cid == 1)
                def _():
                    pltpu.async_copy(zref[1].at[eidx.at[2 * g]],
                                     zrows.at[pl.ds(b * CH, CH)], gsem.at[b])

            def drain_gather(b, h=h, zref=zref):
                pltpu.make_async_copy(es[h].at[pl.ds(0, CH)],
                                      esv.at[pl.ds(b * CH, CH)],
                                      gsem.at[b]).wait()
                pltpu.make_async_copy(ed[h].at[pl.ds(0, CH)],
                                      edv.at[pl.ds(b * CH, CH)],
                                      gsem.at[b]).wait()
                pltpu.make_async_copy(zref[0].at[pl.ds(0, CH)],
                                      zrows.at[pl.ds(b * CH, CH)],
                                      gsem.at[b]).wait()

            def drain_scatter(b, h=h, zref=zref):
                pltpu.make_async_copy(zref[0].at[pl.ds(0, CH)],
                                      zrows.at[pl.ds(b * CH, CH)],
                                      ssem.at[b]).wait()

            def drain_den(b, h=h):
                pltpu.make_async_copy(es[h].at[pl.ds(0, CH)],
                                      exv.at[pl.ds(b * XB, CH)],
                                      dsem.at[b]).wait()

            pltpu.sync_copy(zrows0.at[pl.ds(rbase, RPT)], acc.at[pl.ds(rbase, RPT)])
            pltpu.sync_copy(zvec0.at[pl.ds(rbase, RPT)], dacc.at[pl.ds(rbase, RPT)])
            plsc.subcore_barrier()

            issue_eidx(0, 0)
            issue_eidx(1, 1)
            wait_eidx(0)
            issue(0, 0)

            def chunk(kk, carry, h=h):
                b = lax.rem(kk, 2)
                nb = 1 - b
                g = lax.rem(kk, 3)
                gn = lax.rem(kk + 1, 3)
                g2 = lax.rem(kk + 2, 3)

                @pl.when(kk + 1 < NCHUNK)
                def _():
                    @pl.when(kk >= 1)
                    def _():
                        drain_scatter(nb)

                        @pl.when(cid == dcore)
                        def _():
                            drain_den(nb)

                    @pl.when(kk + 2 < NCHUNK)
                    def _():
                        issue_eidx(kk + 2, g2)

                    wait_eidx(gn)
                    issue(nb, gn)

                drain_gather(b)

                for j in range(CH // 16):
                    t = (esv[pl.ds(b * CH + 16 * j, 16)]
                         + edv[pl.ds(b * CH + 16 * j, 16)])
                    t = jnp.maximum(t, 0.01 * t)
                    exv[pl.ds(b * XB + 16 * j, 16)] = jnp.exp(t)

                @pl.when(cid == dcore)
                def _():
                    pltpu.async_copy(exv.at[pl.ds(b * XB, CH)],
                                     dacc.at[eidx.at[2 * g + 1]], dsem.at[b],
                                     add=True)

                @plsc.parallel_loop(0, CH, unroll=8)
                def scale(e):
                    s = exv[pl.ds(b * XB + e, 16)][0]
                    r = b * CH + e
                    for j in range(8):
                        sl = pl.ds(16 * j, 16)
                        zrows[r, sl] = zrows[r, sl] * s
                pltpu.async_copy(zrows.at[pl.ds(b * CH, CH)],
                                 acc.at[eidx.at[2 * g + 1]], ssem.at[b],
                                 add=True)
                return carry

            lax.fori_loop(0, NCHUNK, chunk, None)
            drain_scatter(0)
            drain_scatter(1)

            @pl.when(cid == dcore)
            def _():
                drain_den(0)
                drain_den(1)

            plsc.subcore_barrier()

            @pl.when(cid == 0)
            def _(h=h):
                pltpu.sync_copy(acc.at[pl.ds(rbase, RPT)],
                                numer[2 * h].at[pl.ds(rbase, RPT)])

            @pl.when(cid == 1)
            def _(h=h):
                pltpu.sync_copy(acc.at[pl.ds(rbase, RPT)],
                                numer[2 * h + 1].at[pl.ds(rbase, RPT)])

            @pl.when(cid == dcore)
            def _(h=h):
                pltpu.sync_copy(dacc.at[pl.ds(rbase, RPT)],
                                den[h].at[pl.ds(rbase, RPT)])

    return pl.kernel(body, out_type=out_type, mesh=_sc_mesh(),
                     scratch_types=scratch)


_edge_pass4 = _make_edge_pass(4)
_edge_pass1 = _make_edge_pass(1)


def _k2_project(h0p, wcat, aall, gidp):
    def kern(h0_ref, w_ref, a_ref, gid_ref, *outs):
        zs = outs[0:8]
        ev = outs[8:16]
        s0_ref, cnt_ref = outs[16], outs[17]
        i = pl.program_id(0)
        h0b = h0_ref[...]
        z = jnp.dot(h0b, w_ref[...], preferred_element_type=F32)
        eall = jnp.dot(z, a_ref[...], preferred_element_type=F32)
        for h in range(4):
            zs[2 * h][...] = z[:, 256 * h:256 * h + 128]
            zs[2 * h + 1][...] = z[:, 256 * h + 128:256 * (h + 1)]
            ev[h][...] = eall[:, 2 * h]
            ev[4 + h][...] = eall[:, 2 * h + 1]
        gi = lax.broadcasted_iota(jnp.int32, (NG, RB), 0)
        gf = (gi == gid_ref[...][None, :]).astype(F32)
        s0c = jnp.dot(gf, h0b, preferred_element_type=F32)
        cntc = jnp.sum(gf, axis=1)

        @pl.when(i == 0)
        def _():
            s0_ref[...] = jnp.zeros_like(s0_ref)
            cnt_ref[...] = jnp.zeros_like(cnt_ref)

        s0_ref[...] += s0c
        cnt_ref[...] += cntc

    zspec = pl.BlockSpec((RB, 128), lambda i: (i, 0))
    vspec = pl.BlockSpec((RB,), lambda i: (i,))
    return pl.pallas_call(
        kern,
        grid=(GRID,),
        in_specs=[
            pl.BlockSpec((RB, H), lambda i: (i, 0)),
            pl.BlockSpec((H, 4 * H), lambda i: (0, 0)),
            pl.BlockSpec((4 * H, 8), lambda i: (0, 0)),
            pl.BlockSpec((RB,), lambda i: (i,)),
        ],
        out_specs=[zspec] * 8 + [vspec] * 8 + [
            pl.BlockSpec((NG, H), lambda i: (0, 0)),
            pl.BlockSpec((NG,), lambda i: (0,)),
        ],
        out_shape=[jax.ShapeDtypeStruct((NP, 128), F32)] * 8
        + [jax.ShapeDtypeStruct((NP,), F32)] * 8
        + [jax.ShapeDtypeStruct((NG, H), F32),
           jax.ShapeDtypeStruct((NG,), F32)],
    )(h0p, wcat, aall, gidp)


def _norm(nums, dens, ssum, ssq, gmat, bmat):
    nh = len(dens)
    pieces = []
    for h in range(nh):
        d = jnp.maximum(dens[h], 1e-16)
        for half in range(2):
            x = jnp.maximum(nums[2 * h + half] / d[:, None], 0.0)
            mu = ssum[2 * h + half, :] * (1.0 / N)
            var = ssq[2 * h + half, :] * (1.0 / N) - mu * mu
            xn = (x - mu[None, :]) * lax.rsqrt(var + 1e-5)[None, :]
            pieces.append(xn * gmat[2 * h + half, :][None, :]
                          + bmat[2 * h + half, :][None, :])
    return jnp.concatenate(pieces, axis=1)


def _k5_mid(nums, dens, wo, ao, gmat, bmat):
    def kern(*refs):
        nrefs = refs[0:8]
        drefs = refs[8:12]
        wo_ref, ao_ref, g_ref, b_ref = refs[12:16]
        zol, zoh, eso, edo = refs[16:20]
        sum_ref, ssq_ref = refs[20:22]
        i = pl.program_id(0)
        ii = lax.rem(i, GRID)

        @pl.when(i < GRID)
        def _():
            ridx = ii * RB + lax.broadcasted_iota(jnp.int32, (RB, 128), 0)
            valid = (ridx < N).astype(F32)
            srows, qrows = [], []
            for h in range(4):
                d = jnp.maximum(drefs[h][...], 1e-16)
                for half in range(2):
                    x = jnp.maximum(nrefs[2 * h + half][...] / d[:, None], 0.0)
                    x = x * valid
                    srows.append(jnp.sum(x, axis=0))
                    qrows.append(jnp.sum(x * x, axis=0))
            scur = jnp.stack(srows, axis=0)
            qcur = jnp.stack(qrows, axis=0)

            @pl.when(i == 0)
            def _():
                sum_ref[...] = jnp.zeros_like(sum_ref)
                ssq_ref[...] = jnp.zeros_like(ssq_ref)

            sum_ref[...] += scur
            ssq_ref[...] += qcur

        @pl.when(i >= GRID)
        def _():
            hcat = _norm([r[...] for r in nrefs], [r[...] for r in drefs],
                         sum_ref[...], ssq_ref[...], g_ref[...], b_ref[...])
            zo = jnp.dot(hcat, wo_ref[...], preferred_element_type=F32)
            eo = jnp.dot(zo, ao_ref[...], preferred_element_type=F32)
            zol[...] = zo[:, :128]
            zoh[...] = zo[:, 128:]
            eso[...] = eo[:, 0]
            edo[...] = eo[:, 1]

    nspec = pl.BlockSpec((RB, 128), lambda i: (lax.rem(i, GRID), 0))
    dspec = pl.BlockSpec((RB,), lambda i: (lax.rem(i, GRID),))
    return pl.pallas_call(
        kern,
        grid=(2 * GRID,),
        in_specs=[nspec] * 8 + [dspec] * 4 + [
            pl.BlockSpec((4 * H, H), lambda i: (0, 0)),
            pl.BlockSpec((H, 2), lambda i: (0, 0)),
            pl.BlockSpec((8, 128), lambda i: (0, 0)),
            pl.BlockSpec((8, 128), lambda i: (0, 0)),
        ],
        out_specs=[nspec, nspec, dspec, dspec],
        out_shape=[jax.ShapeDtypeStruct((NP, 128), F32)] * 2
        + [jax.ShapeDtypeStruct((NP,), F32)] * 2,
        scratch_shapes=[pltpu.VMEM((8, 128), F32), pltpu.VMEM((8, 128), F32)],
    )(*nums, *dens, wo, ao, gmat, bmat)


def _k7_readout(numol, numoh, deno, gmat, bmat, gidp):
    def kern(nl, nh_, dr, g_ref, b_ref, gid_ref, s1_ref, sum_ref, ssq_ref):
        i = pl.program_id(0)
        ii = lax.rem(i, GRID)

        @pl.when(i < GRID)
        def _():
            ridx = ii * RB + lax.broadcasted_iota(jnp.int32, (RB, 128), 0)
            valid = (ridx < N).astype(F32)
            d = jnp.maximum(dr[...], 1e-16)
            srows, qrows = [], []
            for half in range(2):
                x = jnp.maximum([nl, nh_][half][...] / d[:, None], 0.0)
                x = x * valid
                srows.append(jnp.sum(x, axis=0))
                qrows.append(jnp.sum(x * x, axis=0))

            @pl.when(i == 0)
            def _():
                sum_ref[...] = jnp.zeros_like(sum_ref)
                ssq_ref[...] = jnp.zeros_like(ssq_ref)

            sum_ref[...] += jnp.stack(srows, axis=0)
            ssq_ref[...] += jnp.stack(qrows, axis=0)

        @pl.when(i >= GRID)
        def _():
            hfin = _norm([nl[...], nh_[...]], [dr[...]],
                         sum_ref[...], ssq_ref[...], g_ref[...], b_ref[...])
            gi = lax.broadcasted_iota(jnp.int32, (NG, RB), 0)
            gf = (gi == gid_ref[...][None, :]).astype(F32)
            s1c = jnp.dot(gf, hfin, preferred_element_type=F32)

            @pl.when(i == GRID)
            def _():
                s1_ref[...] = jnp.zeros_like(s1_ref)

            s1_ref[...] += s1c

    return pl.pallas_call(
        kern,
        grid=(2 * GRID,),
        in_specs=[
            pl.BlockSpec((RB, 128), lambda i: (lax.rem(i, GRID), 0)),
            pl.BlockSpec((RB, 128), lambda i: (lax.rem(i, GRID), 0)),
            pl.BlockSpec((RB,), lambda i: (lax.rem(i, GRID),)),
            pl.BlockSpec((2, 128), lambda i: (0, 0)),
            pl.BlockSpec((2, 128), lambda i: (0, 0)),
            pl.BlockSpec((RB,), lambda i: (lax.rem(i, GRID),)),
        ],
        out_specs=pl.BlockSpec((NG, H), lambda i: (0, 0)),
        out_shape=jax.ShapeDtypeStruct((NG, H), F32),
        scratch_shapes=[pltpu.VMEM((2, 128), F32), pltpu.VMEM((2, 128), F32)],
    )(numol, numoh, deno, gmat, bmat, gidp)


def _k8_lstm(s0, s1, cnt, p):
    def kern(s0_ref, s1_ref, cnt_ref, wih0, whh0, bih0, bhh0,
             wih1, whh1, bih1, bhh1, out_ref):
        c = jnp.maximum(cnt_ref[...], 1.0)
        x0 = s0_ref[...] / c[:, None]
        x1 = s1_ref[...] / c[:, None]
        inp = [x0, x1]
        wih = [wih0, wih1]
        whh = [whh0, whh1]
        bih = [bih0, bih1]
        bhh = [bhh0, bhh1]
        dn = (((1,), (1,)), ((), ()))
        hfs = []
        for l in range(2):
            h = jnp.zeros((NG, H), F32)
            cc = jnp.zeros((NG, H), F32)
            outs = []
            for t in range(2):
                gates = (lax.dot_general(inp[t], wih[l][...], dn,
                                         preferred_element_type=F32)
                         + bih[l][...][None, :]
                         + lax.dot_general(h, whh[l][...], dn,
                                           preferred_element_type=F32)
                         + bhh[l][...][None, :])
                i_ = jax.nn.sigmoid(gates[:, 0:H])
                f_ = jax.nn.sigmoid(gates[:, H:2 * H])
                g_ = jnp.tanh(gates[:, 2 * H:3 * H])
                o_ = jax.nn.sigmoid(gates[:, 3 * H:4 * H])
                cc = f_ * cc + i_ * g_
                h = o_ * jnp.tanh(cc)
                outs.append(h)
            inp = outs
            hfs.append(h)
        out_ref[...] = hfs[0] + hfs[1]

    return pl.pallas_call(
        kern,
        out_shape=jax.ShapeDtypeStruct((NG, H), F32),
    )(s0, s1, cnt, p['Wih0'], p['Whh0'], p['bih0'], p['bhh0'],
      p['Wih1'], p['Whh1'], p['bih1'], p['bhh1'])


def kernel(params, wid, edge_index, graph_ids):
    p = params
    i32 = jnp.int32
    src = edge_index[0].astype(i32)
    dst = edge_index[1].astype(i32)

    widp = jnp.concatenate([wid.astype(i32), jnp.zeros((NP - N,), i32)])
    gidp = jnp.concatenate([graph_ids.astype(i32),
                            jnp.full((NP - N,), NG + 40, i32)])
    pidx = (N + (jnp.arange(EP - E, dtype=i32) % (NP - N))).astype(i32)
    sd = jnp.stack([jnp.concatenate([src, pidx]),
                    jnp.concatenate([dst, pidx])])
    zrows0 = jnp.zeros((NP, 128), F32)
    zvec0 = jnp.zeros((NP,), F32)

    h0p = _emb_gather(p['emb'], widp)

    wcat = jnp.concatenate([p['W%d' % i] for i in range(4)], axis=1)
    aall = jnp.zeros((4 * H, 8), F32)
    for h in range(4):
        aall = aall.at[256 * h:256 * (h + 1), 2 * h].set(p['a%d' % h][:H])
        aall = aall.at[256 * h:256 * (h + 1), 2 * h + 1].set(p['a%d' % h][H:])

    k2 = _k2_project(h0p, wcat, aall, gidp)
    zs = k2[0:8]
    evs = k2[8:16]
    s0, cnt = k2[16], k2[17]

    ep4 = _edge_pass4(sd, zrows0, zvec0,
                      evs[0], evs[1], evs[2], evs[3],
                      evs[4], evs[5], evs[6], evs[7],
                      zs[0], zs[1], zs[2], zs[3],
                      zs[4], zs[5], zs[6], zs[7])
    nums = ep4[0:8]
    dens = ep4[8:12]

    gmat = jnp.concatenate([p['g%d' % h].reshape(2, 128) for h in range(4)])
    bmat = jnp.concatenate([p['b%d' % h].reshape(2, 128) for h in range(4)])
    ao = jnp.stack([p['ao'][:H], p['ao'][H:]], axis=1)

    zol, zoh, eso, edo = _k5_mid(nums, dens, p['Wo'], ao, gmat, bmat)

    ep1 = _edge_pass1(sd, zrows0, zvec0, eso, edo, zol, zoh)
    numol, numoh, deno = ep1

    s1 = _k7_readout(numol, numoh, deno,
                     p['go'].reshape(2, 128), p['bo'].reshape(2, 128), gidp)

    return _k8_lstm(s0, s1, cnt, p)


# speedup vs baseline: 1.1595x; 1.0099x over previous
import functools

import jax
import jax.numpy as jnp
from jax import lax
from jax.experimental import pallas as pl
from jax.experimental.pallas import tpu as pltpu
from jax.experimental.pallas import tpu_sc as plsc

N = 10000
NP = 10240
E = 160000
H = 256
NG = 256
CH = 128
EPT = 10112
NCHUNK = EPT // CH
EP = EPT * 16
RPT = NP // 16
RB = 512
GRID = NP // RB
F32 = jnp.float32


def _sc_mesh():
    return plsc.VectorSubcoreMesh(core_axis_name="c", subcore_axis_name="s",
                                  num_cores=2, num_subcores=16)


@functools.partial(
    pl.kernel,
    out_type=jax.ShapeDtypeStruct((NP, H), F32),
    mesh=_sc_mesh(),
    scratch_types=[
        pltpu.VMEM((64,), jnp.int32),
        pltpu.VMEM((64, H), F32),
        pltpu.SemaphoreType.DMA,
    ],
)
def _emb_gather(emb_hbm, wid_hbm, out_hbm, idx_v, rows_v, sem):
    wid = lax.axis_index("s") * 2 + lax.axis_index("c")
    base = wid * (NP // 32)

    def body(j, carry):
        b = base + j * 64
        pltpu.sync_copy(wid_hbm.at[pl.ds(b, 64)], idx_v)
        pltpu.async_copy(emb_hbm.at[idx_v], rows_v, sem).wait()
        pltpu.sync_copy(rows_v, out_hbm.at[pl.ds(b, 64)])
        return carry

    lax.fori_loop(0, (NP // 32) // 64, body, None)


def _make_edge_pass(nh):
    out_type = (
        [jax.ShapeDtypeStruct((NP, 128), F32) for _ in range(2 * nh)]
        + [jax.ShapeDtypeStruct((NP,), F32) for _ in range(nh)]
    )
    scratch = [
        pltpu.VMEM((6, CH), jnp.int32),
        pltpu.VMEM((2 * CH,), F32),
        pltpu.VMEM((2 * CH,), F32),
        pltpu.VMEM((2 * (CH + 16),), F32),
        pltpu.VMEM((2 * CH, 128), F32),
        pltpu.VMEM_SHARED((NP, 128), F32),
        pltpu.VMEM_SHARED((NP,), F32),
        pltpu.SemaphoreType.DMA((2,)),
        pltpu.SemaphoreType.DMA((2,)),
        pltpu.SemaphoreType.DMA((2,)),
        pltpu.SemaphoreType.DMA((3,)),
    ]

    def body(*refs):
        sd, zvec0 = refs[0:2]
        es = refs[2:2 + nh]
        ed = refs[2 + nh:2 + 2 * nh]
        zz = refs[2 + 2 * nh:2 + 4 * nh]
        o = 2 + 4 * nh
        numer = refs[o:o + 2 * nh]
        den = refs[o + 2 * nh:o + 3 * nh]
        (eidx, esv, edv, exv, zrows, acc, dacc,
         gsem, ssem, dsem, esem) = refs[o + 3 * nh:]
        XB = CH + 16

        cid = lax.axis_index("c")
        tid = lax.axis_index("s")
        rbase = tid * RPT

        def issue_eidx(kk, g):
            pltpu.async_copy(sd.at[:, pl.ds(tid * EPT + kk * CH, CH)],
                             eidx.at[pl.ds(2 * g, 2)], esem.at[g])

        def wait_eidx(g):
            pltpu.make_async_copy(sd.at[:, pl.ds(0, CH)],
                                  eidx.at[pl.ds(2 * g, 2)], esem.at[g]).wait()

        for h in range(nh):
            zref = [zz[2 * h], zz[2 * h + 1]]
            dcore = h % 2

            def issue(b, g, h=h, zref=zref):
                pltpu.async_copy(es[h].at[eidx.at[2 * g]],
                                 esv.at[pl.ds(b * CH, CH)], gsem.at[b])
                pltpu.async_copy(ed[h].at[eidx.at[2 * g + 1]],
                                 edv.at[pl.ds(b * CH, CH)], gsem.at[b])

                @pl.when(cid == 0)
                def _():
                    pltpu.async_copy(zref[0].at[eidx.at[2 * g]],
                                     zrows.at[pl.ds(b * CH, CH)], gsem.at[b])

                @pl.when(cid == 1)
                def _():
                    pltpu.async_copy(zref[1].at[eidx.at[2 * g]],
                                     zrows.at[pl.ds(b * CH, CH)], gsem.at[b])

            def drain_gather(b, h=h, zref=zref):
                pltpu.make_async_copy(es[h].at[pl.ds(0, CH)],
                                      esv.at[pl.ds(b * CH, CH)],
                                      gsem.at[b]).wait()
                pltpu.make_async_copy(ed[h].at[pl.ds(0, CH)],
                                      edv.at[pl.ds(b * CH, CH)],
                                      gsem.at[b]).wait()
                pltpu.make_async_copy(zref[0].at[pl.ds(0, CH)],
                                      zrows.at[pl.ds(b * CH, CH)],
                                      gsem.at[b]).wait()

            def drain_scatter(b, h=h, zref=zref):
                pltpu.make_async_copy(zref[0].at[pl.ds(0, CH)],
                                      zrows.at[pl.ds(b * CH, CH)],
                                      ssem.at[b]).wait()

            def drain_den(b, h=h):
                pltpu.make_async_copy(es[h].at[pl.ds(0, CH)],
                                      exv.at[pl.ds(b * XB, CH)],
                                      dsem.at[b]).wait()

            def zfill(r, carry):
                for j in range(8):
                    zrows[r, pl.ds(16 * j, 16)] = jnp.zeros((16,), F32)
                return carry

            lax.fori_loop(0, 2 * CH, zfill, None)
            pltpu.sync_copy(zrows, acc.at[pl.ds(rbase, 2 * CH)])
            pltpu.sync_copy(zrows, acc.at[pl.ds(rbase + 2 * CH, 2 * CH)])
            pltpu.sync_copy(zrows.at[pl.ds(0, CH)],
                            acc.at[pl.ds(rbase + 4 * CH, CH)])
            pltpu.sync_copy(zvec0.at[pl.ds(rbase, RPT)], dacc.at[pl.ds(rbase, RPT)])
            plsc.subcore_barrier()

            issue_eidx(0, 0)
            issue_eidx(1, 1)
            wait_eidx(0)
            issue(0, 0)

            def chunk(kk, carry, h=h):
                b = lax.rem(kk, 2)
                nb = 1 - b
                g = lax.rem(kk, 3)
                gn = lax.rem(kk + 1, 3)
                g2 = lax.rem(kk + 2, 3)

                @pl.when(kk + 1 < NCHUNK)
                def _():
                    @pl.when(kk >= 1)
                    def _():
                        drain_scatter(nb)

                        @pl.when(cid == dcore)
                        def _():
                            drain_den(nb)

                    @pl.when(kk + 2 < NCHUNK)
                    def _():
                        issue_eidx(kk + 2, g2)

                    wait_eidx(gn)
                    issue(nb, gn)

                drain_gather(b)

                for j in range(CH // 16):
                    t = (esv[pl.ds(b * CH + 16 * j, 16)]
                         + edv[pl.ds(b * CH + 16 * j, 16)])
                    t = jnp.maximum(t, 0.01 * t)
                    exv[pl.ds(b * XB + 16 * j, 16)] = jnp.exp(t)

                @pl.when(cid == dcore)
                def _():
                    pltpu.async_copy(exv.at[pl.ds(b * XB, CH)],
                                     dacc.at[eidx.at[2 * g + 1]], dsem.at[b],
                                     add=True)

                @plsc.parallel_loop(0, CH, unroll=8)
                def scale(e):
                    s = exv[pl.ds(b * XB + e, 16)][0]
                    r = b * CH + e
                    for j in range(8):
                        sl = pl.ds(16 * j, 16)
                        zrows[r, sl] = zrows[r, sl] * s
                pltpu.async_copy(zrows.at[pl.ds(b * CH, CH)],
                                 acc.at[eidx.at[2 * g + 1]], ssem.at[b],
                                 add=True)
                return carry

            lax.fori_loop(0, NCHUNK, chunk, None)
            drain_scatter(0)
            drain_scatter(1)

            @pl.when(cid == dcore)
            def _():
                drain_den(0)
                drain_den(1)

            plsc.subcore_barrier()

            @pl.when(cid == 0)
            def _(h=h):
                pltpu.sync_copy(acc.at[pl.ds(rbase, RPT)],
                                numer[2 * h].at[pl.ds(rbase, RPT)])

            @pl.when(cid == 1)
            def _(h=h):
                pltpu.sync_copy(acc.at[pl.ds(rbase, RPT)],
                                numer[2 * h + 1].at[pl.ds(rbase, RPT)])

            @pl.when(cid == dcore)
            def _(h=h):
                pltpu.sync_copy(dacc.at[pl.ds(rbase, RPT)],
                                den[h].at[pl.ds(rbase, RPT)])

    return pl.kernel(body, out_type=out_type, mesh=_sc_mesh(),
                     scratch_types=scratch)


_edge_pass4 = _make_edge_pass(4)
_edge_pass1 = _make_edge_pass(1)


def _k2_project(h0p, wcat, aall, gidp):
    def kern(h0_ref, w_ref, a_ref, gid_ref, *outs):
        zs = outs[0:8]
        ev = outs[8:16]
        s0_ref, cnt_ref = outs[16], outs[17]
        i = pl.program_id(0)
        h0b = h0_ref[...]
        z = jnp.dot(h0b, w_ref[...], preferred_element_type=F32)
        eall = jnp.dot(z, a_ref[...], preferred_element_type=F32)
        for h in range(4):
            zs[2 * h][...] = z[:, 256 * h:256 * h + 128]
            zs[2 * h + 1][...] = z[:, 256 * h + 128:256 * (h + 1)]
            ev[h][...] = eall[:, 2 * h]
            ev[4 + h][...] = eall[:, 2 * h + 1]
        gi = lax.broadcasted_iota(jnp.int32, (NG, RB), 0)
        gf = (gi == gid_ref[...][None, :]).astype(F32)
        s0c = jnp.dot(gf, h0b, preferred_element_type=F32)
        cntc = jnp.sum(gf, axis=1)

        @pl.when(i == 0)
        def _():
            s0_ref[...] = jnp.zeros_like(s0_ref)
            cnt_ref[...] = jnp.zeros_like(cnt_ref)

        s0_ref[...] += s0c
        cnt_ref[...] += cntc

    zspec = pl.BlockSpec((RB, 128), lambda i: (i, 0))
    vspec = pl.BlockSpec((RB,), lambda i: (i,))
    return pl.pallas_call(
        kern,
        grid=(GRID,),
        in_specs=[
            pl.BlockSpec((RB, H), lambda i: (i, 0)),
            pl.BlockSpec((H, 4 * H), lambda i: (0, 0)),
            pl.BlockSpec((4 * H, 8), lambda i: (0, 0)),
            pl.BlockSpec((RB,), lambda i: (i,)),
        ],
        out_specs=[zspec] * 8 + [vspec] * 8 + [
            pl.BlockSpec((NG, H), lambda i: (0, 0)),
            pl.BlockSpec((NG,), lambda i: (0,)),
        ],
        out_shape=[jax.ShapeDtypeStruct((NP, 128), F32)] * 8
        + [jax.ShapeDtypeStruct((NP,), F32)] * 8
        + [jax.ShapeDtypeStruct((NG, H), F32),
           jax.ShapeDtypeStruct((NG,), F32)],
    )(h0p, wcat, aall, gidp)


def _norm(nums, dens, ssum, ssq, gmat, bmat):
    nh = len(dens)
    pieces = []
    for h in range(nh):
        d = jnp.maximum(dens[h], 1e-16)
        for half in range(2):
            x = jnp.maximum(nums[2 * h + half] / d[:, None], 0.0)
            mu = ssum[2 * h + half, :] * (1.0 / N)
            var = ssq[2 * h + half, :] * (1.0 / N) - mu * mu
            xn = (x - mu[None, :]) * lax.rsqrt(var + 1e-5)[None, :]
            pieces.append(xn * gmat[2 * h + half, :][None, :]
                          + bmat[2 * h + half, :][None, :])
    return jnp.concatenate(pieces, axis=1)


def _k5_mid(nums, dens, wo, ao, gmat, bmat):
    def kern(*refs):
        nrefs = refs[0:8]
        drefs = refs[8:12]
        wo_ref, ao_ref, g_ref, b_ref = refs[12:16]
        zol, zoh, eso, edo = refs[16:20]
        sum_ref, ssq_ref = refs[20:22]
        i = pl.program_id(0)
        ii = lax.rem(i, GRID)

        @pl.when(i < GRID)
        def _():
            ridx = ii * RB + lax.broadcasted_iota(jnp.int32, (RB, 128), 0)
            valid = (ridx < N).astype(F32)
            srows, qrows = [], []
            for h in range(4):
                d = jnp.maximum(drefs[h][...], 1e-16)
                for half in range(2):
                    x = jnp.maximum(nrefs[2 * h + half][...] / d[:, None], 0.0)
                    x = x * valid
                    srows.append(jnp.sum(x, axis=0))
                    qrows.append(jnp.sum(x * x, axis=0))
            scur = jnp.stack(srows, axis=0)
            qcur = jnp.stack(qrows, axis=0)

            @pl.when(i == 0)
            def _():
                sum_ref[...] = jnp.zeros_like(sum_ref)
                ssq_ref[...] = jnp.zeros_like(ssq_ref)

            sum_ref[...] += scur
            ssq_ref[...] += qcur

        @pl.when(i >= GRID)
        def _():
            hcat = _norm([r[...] for r in nrefs], [r[...] for r in drefs],
                         sum_ref[...], ssq_ref[...], g_ref[...], b_ref[...])
            zo = jnp.dot(hcat, wo_ref[...], preferred_element_type=F32)
            eo = jnp.dot(zo, ao_ref[...], preferred_element_type=F32)
            zol[...] = zo[:, :128]
            zoh[...] = zo[:, 128:]
            eso[...] = eo[:, 0]
            edo[...] = eo[:, 1]

    nspec = pl.BlockSpec((RB, 128), lambda i: (lax.rem(i, GRID), 0))
    dspec = pl.BlockSpec((RB,), lambda i: (lax.rem(i, GRID),))
    return pl.pallas_call(
        kern,
        grid=(2 * GRID,),
        in_specs=[nspec] * 8 + [dspec] * 4 + [
            pl.BlockSpec((4 * H, H), lambda i: (0, 0)),
            pl.BlockSpec((H, 2), lambda i: (0, 0)),
            pl.BlockSpec((8, 128), lambda i: (0, 0)),
            pl.BlockSpec((8, 128), lambda i: (0, 0)),
        ],
        out_specs=[nspec, nspec, dspec, dspec],
        out_shape=[jax.ShapeDtypeStruct((NP, 128), F32)] * 2
        + [jax.ShapeDtypeStruct((NP,), F32)] * 2,
        scratch_shapes=[pltpu.VMEM((8, 128), F32), pltpu.VMEM((8, 128), F32)],
    )(*nums, *dens, wo, ao, gmat, bmat)


def _k7_readout(numol, numoh, deno, gmat, bmat, gidp):
    def kern(nl, nh_, dr, g_ref, b_ref, gid_ref, s1_ref, sum_ref, ssq_ref):
        i = pl.program_id(0)
        ii = lax.rem(i, GRID)

        @pl.when(i < GRID)
        def _():
            ridx = ii * RB + lax.broadcasted_iota(jnp.int32, (RB, 128), 0)
            valid = (ridx < N).astype(F32)
            d = jnp.maximum(dr[...], 1e-16)
            srows, qrows = [], []
            for half in range(2):
                x = jnp.maximum([nl, nh_][half][...] / d[:, None], 0.0)
                x = x * valid
                srows.append(jnp.sum(x, axis=0))
                qrows.append(jnp.sum(x * x, axis=0))

            @pl.when(i == 0)
            def _():
                sum_ref[...] = jnp.zeros_like(sum_ref)
                ssq_ref[...] = jnp.zeros_like(ssq_ref)

            sum_ref[...] += jnp.stack(srows, axis=0)
            ssq_ref[...] += jnp.stack(qrows, axis=0)

        @pl.when(i >= GRID)
        def _():
            hfin = _norm([nl[...], nh_[...]], [dr[...]],
                         sum_ref[...], ssq_ref[...], g_ref[...], b_ref[...])
            gi = lax.broadcasted_iota(jnp.int32, (NG, RB), 0)
            gf = (gi == gid_ref[...][None, :]).astype(F32)
            s1c = jnp.dot(gf, hfin, preferred_element_type=F32)

            @pl.when(i == GRID)
            def _():
                s1_ref[...] = jnp.zeros_like(s1_ref)

            s1_ref[...] += s1c

    return pl.pallas_call(
        kern,
        grid=(2 * GRID,),
        in_specs=[
            pl.BlockSpec((RB, 128), lambda i: (lax.rem(i, GRID), 0)),
            pl.BlockSpec((RB, 128), lambda i: (lax.rem(i, GRID), 0)),
            pl.BlockSpec((RB,), lambda i: (lax.rem(i, GRID),)),
            pl.BlockSpec((2, 128), lambda i: (0, 0)),
            pl.BlockSpec((2, 128), lambda i: (0, 0)),
            pl.BlockSpec((RB,), lambda i: (lax.rem(i, GRID),)),
        ],
        out_specs=pl.BlockSpec((NG, H), lambda i: (0, 0)),
        out_shape=jax.ShapeDtypeStruct((NG, H), F32),
        scratch_shapes=[pltpu.VMEM((2, 128), F32), pltpu.VMEM((2, 128), F32)],
    )(numol, numoh, deno, gmat, bmat, gidp)


def _k8_lstm(s0, s1, cnt, p):
    def kern(s0_ref, s1_ref, cnt_ref, wih0, whh0, bih0, bhh0,
             wih1, whh1, bih1, bhh1, out_ref):
        c = jnp.maximum(cnt_ref[...], 1.0)
        x0 = s0_ref[...] / c[:, None]
        x1 = s1_ref[...] / c[:, None]
        inp = [x0, x1]
        wih = [wih0, wih1]
        whh = [whh0, whh1]
        bih = [bih0, bih1]
        bhh = [bhh0, bhh1]
        dn = (((1,), (1,)), ((), ()))
        hfs = []
        for l in range(2):
            h = jnp.zeros((NG, H), F32)
            cc = jnp.zeros((NG, H), F32)
            outs = []
            for t in range(2):
                gates = (lax.dot_general(inp[t], wih[l][...], dn,
                                         preferred_element_type=F32)
                         + bih[l][...][None, :]
                         + lax.dot_general(h, whh[l][...], dn,
                                           preferred_element_type=F32)
                         + bhh[l][...][None, :])
                i_ = jax.nn.sigmoid(gates[:, 0:H])
                f_ = jax.nn.sigmoid(gates[:, H:2 * H])
                g_ = jnp.tanh(gates[:, 2 * H:3 * H])
                o_ = jax.nn.sigmoid(gates[:, 3 * H:4 * H])
                cc = f_ * cc + i_ * g_
                h = o_ * jnp.tanh(cc)
                outs.append(h)
            inp = outs
            hfs.append(h)
        out_ref[...] = hfs[0] + hfs[1]

    return pl.pallas_call(
        kern,
        out_shape=jax.ShapeDtypeStruct((NG, H), F32),
    )(s0, s1, cnt, p['Wih0'], p['Whh0'], p['bih0'], p['bhh0'],
      p['Wih1'], p['Whh1'], p['bih1'], p['bhh1'])


def kernel(params, wid, edge_index, graph_ids):
    p = params
    i32 = jnp.int32
    src = edge_index[0].astype(i32)
    dst = edge_index[1].astype(i32)

    widp = jnp.concatenate([wid.astype(i32), jnp.zeros((NP - N,), i32)])
    gidp = jnp.concatenate([graph_ids.astype(i32),
                            jnp.full((NP - N,), NG + 40, i32)])
    pidx = (N + (jnp.arange(EP - E, dtype=i32) % (NP - N))).astype(i32)
    sd = jnp.stack([jnp.concatenate([src, pidx]),
                    jnp.concatenate([dst, pidx])])
    zvec0 = jnp.zeros((NP,), F32)

    h0p = _emb_gather(p['emb'], widp)

    wcat = jnp.concatenate([p['W%d' % i] for i in range(4)], axis=1)
    aall = jnp.zeros((4 * H, 8), F32)
    for h in range(4):
        aall = aall.at[256 * h:256 * (h + 1), 2 * h].set(p['a%d' % h][:H])
        aall = aall.at[256 * h:256 * (h + 1), 2 * h + 1].set(p['a%d' % h][H:])

    k2 = _k2_project(h0p, wcat, aall, gidp)
    zs = k2[0:8]
    evs = k2[8:16]
    s0, cnt = k2[16], k2[17]

    ep4 = _edge_pass4(sd, zvec0,
                      evs[0], evs[1], evs[2], evs[3],
                      evs[4], evs[5], evs[6], evs[7],
                      zs[0], zs[1], zs[2], zs[3],
                      zs[4], zs[5], zs[6], zs[7])
    nums = ep4[0:8]
    dens = ep4[8:12]

    gmat = jnp.concatenate([p['g%d' % h].reshape(2, 128) for h in range(4)])
    bmat = jnp.concatenate([p['b%d' % h].reshape(2, 128) for h in range(4)])
    ao = jnp.stack([p['ao'][:H], p['ao'][H:]], axis=1)

    zol, zoh, eso, edo = _k5_mid(nums, dens, p['Wo'], ao, gmat, bmat)

    ep1 = _edge_pass1(sd, zvec0, eso, edo, zol, zoh)
    numol, numoh, deno = ep1

    s1 = _k7_readout(numol, numoh, deno,
                     p['go'].reshape(2, 128), p['bo'].reshape(2, 128), gidp)

    return _k8_lstm(s0, s1, cnt, p)
